# baseline (speedup 1.0000x reference)
import functools
import jax
import jax.numpy as jnp
import numpy as np
from jax import lax
from jax.experimental import pallas as pl
from jax.experimental.pallas import tpu as pltpu
from jax.experimental.pallas import tpu_sc as plsc

N = 19881
NT = 16
NPAD = 19968
PT = NPAD // NT
PV = PT // 16
M = 2000
MP = 2048
ST = MP // NT
SV = ST // 16
POST = 300
THRESH = 0.7
CLIP = 46.0
KEY_INVALID = np.uint32(0xFFFFFFFF)
Y2_INVALID = np.int32(0x7F800000)
DUMP = NPAD


def _iota16():
    return lax.iota(jnp.int32, 16)


def _extract(vec, lane):
    return jnp.sum(jnp.where(_iota16() == lane, vec, jnp.zeros((16,), vec.dtype)))


def _sc_body(sc_hbm, dl_hbm, bx_hbm, op_hbm, os_hbm,
             in_s, in_d, in_b,
             lx1, ly1, lw, lh, lx2, ly2, lar, lsc, lkey, lval,
             kv, vv, rankv, destv, hist, ha, gbase, tbase,
             csk, csv, px1g, py1g, pwg, phg, px2g, py2g, parg, pscg,
             x1s, y1s, x2s, y2s, denv, alivev, idxsv,
             cpx1, cpy1, cpw, cph, cps,
             fx1, fy1, fx2, fy2, fden, fidx,
             keepb, tv16, pbuf, sbuf,
             CK, CV, NK, NV, X1, Y1, WW, HH, X2, Y2, AR, SS,
             HIST, HISTS, CNTS, PX1, PY1, PW, PH, PX2, PY2, PAR, PS,
             SKA, SVA, SKB, SVB,
             X1S, Y1S, X2S, Y2S, CHMA,
             sem):
    cid = lax.axis_index("c")
    wid = lax.axis_index("s")
    base = wid * PT

    TAIL = N - 15 * PT

    @pl.when(wid < 15)
    def _():
        pltpu.sync_copy(sc_hbm.at[pl.ds(base, PT)], in_s)
        pltpu.sync_copy(dl_hbm.at[pl.ds(base * 4, PT * 4)], in_d)
        pltpu.sync_copy(bx_hbm.at[pl.ds(base * 4, PT * 4)], in_b)

    @pl.when(wid == 15)
    def _():
        b15 = 15 * PT
        pltpu.sync_copy(sc_hbm.at[pl.ds(b15, TAIL)], in_s.at[pl.ds(0, TAIL)])
        pltpu.sync_copy(dl_hbm.at[pl.ds(b15 * 4, TAIL * 4)],
                        in_d.at[pl.ds(0, TAIL * 4)])
        pltpu.sync_copy(bx_hbm.at[pl.ds(b15 * 4, TAIL * 4)],
                        in_b.at[pl.ds(0, TAIL * 4)])

    def a_body(i, _):
        o = i * 16
        lanes = _iota16()
        q = (o + lanes) * 4
        d0 = plsc.load_gather(in_d, [q])
        d1 = plsc.load_gather(in_d, [q + 1])
        d2 = plsc.load_gather(in_d, [q + 2])
        d3 = plsc.load_gather(in_d, [q + 3])
        b0 = plsc.load_gather(in_b, [q])
        b1 = plsc.load_gather(in_b, [q + 1])
        b2 = plsc.load_gather(in_b, [q + 2])
        b3 = plsc.load_gather(in_b, [q + 3])
        s = in_s[pl.ds(o, 16)]
        ax = d0 * b0 + b0
        ay = d1 * b1 + b1
        aw = jnp.exp(d2) * b2
        ah = jnp.exp(d3) * b3
        zero = jnp.float32(0.0)
        ax = jnp.maximum(ax, zero)
        ay = jnp.maximum(ay, zero)
        aw = jnp.maximum(aw, zero)
        ah = jnp.maximum(ah, zero)
        x1 = jnp.minimum(ax, CLIP)
        y1 = jnp.minimum(ay, CLIP)
        x2 = jnp.minimum(ax + aw - 1.0, CLIP)
        y2 = jnp.minimum(ay + ah - 1.0, CLIP)
        w_ = x2 - x1 + 1.0
        h_ = y2 - y1 + 1.0
        e = base + o + lanes
        keep = (w_ >= 2.0) & (h_ >= 2.0) & (e < N)
        u = plsc.bitcast(s + zero, jnp.uint32)
        asc = jnp.where((u >> 31) == jnp.uint32(1), ~u, u | jnp.uint32(0x80000000))
        keyd = jnp.where(keep, ~asc, KEY_INVALID)
        lx1[pl.ds(o, 16)] = x1
        ly1[pl.ds(o, 16)] = y1
        lw[pl.ds(o, 16)] = w_
        lh[pl.ds(o, 16)] = h_
        lx2[pl.ds(o, 16)] = x2
        ly2[pl.ds(o, 16)] = y2
        lar[pl.ds(o, 16)] = w_ * h_
        lsc[pl.ds(o, 16)] = s
        lkey[pl.ds(o, 16)] = plsc.bitcast(keyd, jnp.int32)
        lval[pl.ds(o, 16)] = e
        return _

    lax.fori_loop(0, PV, a_body, 0)
    pltpu.sync_copy(lx1, X1.at[pl.ds(base, PT)])
    pltpu.sync_copy(ly1, Y1.at[pl.ds(base, PT)])
    pltpu.sync_copy(lw, WW.at[pl.ds(base, PT)])
    pltpu.sync_copy(lh, HH.at[pl.ds(base, PT)])
    pltpu.sync_copy(lx2, X2.at[pl.ds(base, PT)])
    pltpu.sync_copy(ly2, Y2.at[pl.ds(base, PT)])
    pltpu.sync_copy(lar, AR.at[pl.ds(base, PT)])
    pltpu.sync_copy(lsc, SS.at[pl.ds(base, PT)])

    P = jnp.uint32(0)
    R = jnp.int32(M)
    for rnd, shift in enumerate((24, 16, 8, 0)):
        dmask = jnp.uint32((0xFFFFFFFF00000000 >> (8 * rnd)) & 0xFFFFFFFF)
        hb = rnd * 4096

        for j in range(16):
            hist[pl.ds(j * 16, 16)] = jnp.zeros((16,), jnp.int32)

        def s_hist(i, _, shift=shift, dmask=dmask, P=P):
            o = i * 16
            ku = plsc.bitcast(lkey[pl.ds(o, 16)], jnp.uint32)
            m = (ku & dmask) == (P & dmask)
            d = ((ku >> jnp.uint32(shift)) & jnp.uint32(255)).astype(jnp.int32)
            cnt, last = plsc.scan_count(d, mask=m)
            plsc.addupdate_scatter(hist, [d], cnt, mask=last)
            return _

        lax.fori_loop(0, PV, s_hist, 0)
        pltpu.sync_copy(hist, HISTS.at[pl.ds(hb + wid * 256, 256)])
        plsc.subcore_barrier()
        pltpu.sync_copy(HISTS.at[pl.ds(hb, 4096)], ha)

        def s_find(j, carry):
            found, bsel, rminus, cumbefore = carry
            o = j * 16
            tot = jnp.zeros((16,), jnp.int32)
            for t in range(16):
                tot = tot + ha[pl.ds(t * 256 + o, 16)]
            incl = plsc.cumsum(tot)
            cum = cumbefore + incl
            l = jnp.max(plsc.all_reduce_ffs(cum >= R))
            this = (l < 16) & jnp.logical_not(found)
            bsel = jnp.where(this, o + l, bsel)
            rminus = jnp.where(this,
                               cumbefore + _extract(incl, l) - _extract(tot, l),
                               rminus)
            return (found | (l < 16), bsel, rminus,
                    cumbefore + jnp.sum(tot))

        _, bsel, rminus, _ = lax.fori_loop(
            0, 16, s_find,
            (jnp.bool_(False), jnp.int32(0), jnp.int32(0), jnp.int32(0)))
        P = P | (bsel.astype(jnp.uint32) << jnp.uint32(shift))
        R = R - rminus
    T = P

    def cnt_body(i, carry):
        nlt, nt = carry
        o = i * 16
        ku = plsc.bitcast(lkey[pl.ds(o, 16)], jnp.uint32)
        one = jnp.full((16,), 1, jnp.int32)
        z = jnp.zeros((16,), jnp.int32)
        return (nlt + jnp.sum(jnp.where(ku < T, one, z)),
                nt + jnp.sum(jnp.where(ku == T, one, z)))

    nlt, ntie = lax.fori_loop(0, PV, cnt_body, (jnp.int32(0), jnp.int32(0)))
    lanes0 = _iota16()
    tv16[...] = (jnp.where(lanes0 == 0, nlt, jnp.zeros((16,), jnp.int32))
                 + jnp.where(lanes0 == 1, ntie, jnp.zeros((16,), jnp.int32)))
    pltpu.sync_copy(tv16, CNTS.at[pl.ds(wid * 16, 16)])
    plsc.subcore_barrier()
    pltpu.sync_copy(CNTS, ha.at[pl.ds(0, 256)])

    def base_body(t, carry):
        blt, btie, c1 = carry
        row = ha[pl.ds(t * 16, 16)]
        nlt_t = _extract(row, 0)
        nt_t = _extract(row, 1)
        zero = jnp.int32(0)
        return (blt + jnp.where(t < wid, nlt_t, zero),
                btie + jnp.where(t < wid, nt_t, zero),
                c1 + nlt_t)

    blt, btie, c1 = lax.fori_loop(0, 16, base_body,
                                  (jnp.int32(0), jnp.int32(0), jnp.int32(0)))

    def dest_rows():
        def d2_body(i, carry):
            lt_run, tie_run = carry
            o = i * 16
            ku = plsc.bitcast(lkey[pl.ds(o, 16)], jnp.uint32)
            mlt = ku < T
            meq = ku == T
            one = jnp.full((16,), 1, jnp.int32)
            z = jnp.zeros((16,), jnp.int32)
            ilt = jnp.where(mlt, one, z)
            ieq = jnp.where(meq, one, z)
            ex_lt = plsc.cumsum(ilt) - ilt
            ex_tie = plsc.cumsum(ieq) - ieq
            d_lt = blt + lt_run + ex_lt
            g = btie + tie_run + ex_tie
            d_tie = c1 + g
            dump = MP + ((o + _iota16()) % 128)
            dest = jnp.where(mlt, d_lt,
                             jnp.where(meq & (g < R), d_tie, dump))
            r_ = i // 8
            destv[r_, pl.ds((i % 8) * 16, 16)] = dest
            return lt_run + jnp.sum(ilt), tie_run + jnp.sum(ieq)

        lax.fori_loop(0, PV, d2_body, (jnp.int32(0), jnp.int32(0)))
        for i in range(PV, 80):
            destv[i // 8, pl.ds((i % 8) * 16, 16)] = MP + (i % 8) * 16 + _iota16()

    dest_rows()
    for r in range(10):
        pltpu.async_copy(lkey.at[pl.ds(r * 128, 128)],
                         SKA.at[destv.at[r]], sem).wait()
        pltpu.async_copy(lval.at[pl.ds(r * 128, 128)],
                         SVA.at[destv.at[r]], sem).wait()

    @pl.when(wid == 0)
    def _():
        for j in range(3):
            tv16[...] = jnp.full((16,), -1, jnp.int32)
            pltpu.sync_copy(tv16, SKA.at[pl.ds(M + j * 16, 16)])
            tv16[...] = jnp.zeros((16,), jnp.int32)
            pltpu.sync_copy(tv16, SVA.at[pl.ds(M + j * 16, 16)])

    plsc.subcore_barrier()

    def radix_pass(src_k, src_v, dst_k, dst_v, shift, nv, per_tile, nrows):
        b0 = wid * per_tile
        pltpu.sync_copy(src_k.at[pl.ds(b0, per_tile)], kv.at[pl.ds(0, per_tile)])
        pltpu.sync_copy(src_v.at[pl.ds(b0, per_tile)], vv.at[pl.ds(0, per_tile)])
        for j in range(16):
            hist[pl.ds(j * 16, 16)] = jnp.zeros((16,), jnp.int32)

        def h_body(i, _):
            o = i * 16
            ku = plsc.bitcast(kv[pl.ds(o, 16)], jnp.uint32)
            d = ((ku >> shift) & jnp.uint32(255)).astype(jnp.int32)
            pre = plsc.load_gather(hist, [d])
            cnt, last = plsc.scan_count(d)
            rankv[pl.ds(o, 16)] = pre + cnt - 1
            plsc.addupdate_scatter(hist, [d], cnt, mask=last)
            return _

        lax.fori_loop(0, nv, h_body, 0)
        pltpu.sync_copy(hist, HIST.at[pl.ds(wid * 256, 256)])
        plsc.subcore_barrier()
        pltpu.sync_copy(HIST, ha)

        def g_body(j, carry):
            o = j * 16
            tot = jnp.zeros((16,), jnp.int32)
            tb = jnp.zeros((16,), jnp.int32)
            for t in range(16):
                row = ha[pl.ds(t * 256 + o, 16)]
                tot = tot + row
                tb = tb + jnp.where(jnp.int32(t) < wid, row,
                                    jnp.zeros((16,), jnp.int32))
            incl = plsc.cumsum(tot)
            gbase[pl.ds(o, 16)] = incl - tot + carry
            tbase[pl.ds(o, 16)] = tb
            return carry + jnp.sum(tot)

        lax.fori_loop(0, 16, g_body, jnp.int32(0))

        for r in range(nrows):
            vlo = r * 8
            vhi = min(r * 8 + 8, nv)

            def d_body(i, _, vlo=vlo, r=r):
                o = i * 16
                ku = plsc.bitcast(kv[pl.ds(vlo * 16 + o, 16)], jnp.uint32)
                d = ((ku >> shift) & jnp.uint32(255)).astype(jnp.int32)
                gb = plsc.load_gather(gbase, [d])
                tb = plsc.load_gather(tbase, [d])
                dst = gb + tb + rankv[pl.ds(vlo * 16 + o, 16)]
                destv[r, pl.ds(o, 16)] = dst
                return _

            lax.fori_loop(0, vhi - vlo, d_body, 0)
            for i in range(vhi - vlo, 8):
                destv[r, pl.ds(i * 16, 16)] = DUMP + i * 16 + _iota16()
        for r in range(nrows):
            pltpu.async_copy(kv.at[pl.ds(r * 128, 128)],
                             dst_k.at[destv.at[r]], sem).wait()
            pltpu.async_copy(vv.at[pl.ds(r * 128, 128)],
                             dst_v.at[destv.at[r]], sem).wait()
        plsc.subcore_barrier()

    radix_pass(SKA, SVA, SKB, SVB, jnp.uint32(0), SV, ST, 1)
    radix_pass(SKB, SVB, SKA, SVA, jnp.uint32(8), SV, ST, 1)
    radix_pass(SKA, SVA, SKB, SVB, jnp.uint32(16), SV, ST, 1)
    radix_pass(SKB, SVB, SKA, SVA, jnp.uint32(24), SV, ST, 1)

    sbase = wid * ST
    pltpu.sync_copy(SKA.at[pl.ds(sbase, ST)], csk)
    pltpu.sync_copy(SVA.at[pl.ds(sbase, ST)], csv)
    g1 = pltpu.make_async_copy(X1.at[csv], px1g, sem)
    g2 = pltpu.make_async_copy(Y1.at[csv], py1g, sem)
    g3 = pltpu.make_async_copy(WW.at[csv], pwg, sem)
    g4 = pltpu.make_async_copy(HH.at[csv], phg, sem)
    g5 = pltpu.make_async_copy(X2.at[csv], px2g, sem)
    g6 = pltpu.make_async_copy(Y2.at[csv], py2g, sem)
    g7 = pltpu.make_async_copy(AR.at[csv], parg, sem)
    g8 = pltpu.make_async_copy(SS.at[csv], pscg, sem)
    for g in (g1, g2, g3, g4, g5, g6, g7, g8):
        g.start()
    for g in (g1, g2, g3, g4, g5, g6, g7, g8):
        g.wait()
    pltpu.sync_copy(px1g, PX1.at[pl.ds(sbase, ST)])
    pltpu.sync_copy(py1g, PY1.at[pl.ds(sbase, ST)])
    pltpu.sync_copy(pwg, PW.at[pl.ds(sbase, ST)])
    pltpu.sync_copy(phg, PH.at[pl.ds(sbase, ST)])
    pltpu.sync_copy(px2g, PX2.at[pl.ds(sbase, ST)])
    pltpu.sync_copy(py2g, PY2.at[pl.ds(sbase, ST)])
    pltpu.sync_copy(parg, PAR.at[pl.ds(sbase, ST)])
    pltpu.sync_copy(pscg, PS.at[pl.ds(sbase, ST)])

    def y_body(i, _):
        o = i * 16
        pos = sbase + o + _iota16()
        kk = plsc.bitcast(csk[pl.ds(o, 16)], jnp.uint32)
        valid = (kk != KEY_INVALID) & (pos < M)
        y2b = plsc.bitcast(py2g[pl.ds(o, 16)], jnp.int32)
        kv[pl.ds(o, 16)] = jnp.where(valid, y2b,
                                     jnp.full((16,), Y2_INVALID, jnp.int32))
        vv[pl.ds(o, 16)] = pos
        return _

    lax.fori_loop(0, SV, y_body, 0)
    pltpu.sync_copy(kv.at[pl.ds(0, ST)], CK.at[pl.ds(sbase, ST)])
    pltpu.sync_copy(vv.at[pl.ds(0, ST)], CV.at[pl.ds(sbase, ST)])
    plsc.subcore_barrier()

    radix_pass(CK, CV, NK, NV, jnp.uint32(0), SV, ST, 1)
    radix_pass(NK, NV, CK, CV, jnp.uint32(8), SV, ST, 1)
    radix_pass(CK, CV, NK, NV, jnp.uint32(16), SV, ST, 1)
    radix_pass(NK, NV, CK, CV, jnp.uint32(24), SV, ST, 1)

    pltpu.sync_copy(CK.at[pl.ds(sbase, ST)], csk)
    pltpu.sync_copy(CV.at[pl.ds(sbase, ST)], idxsv)
    pltpu.sync_copy(PX1, cpx1)
    pltpu.sync_copy(PY1, cpy1)
    pltpu.sync_copy(PX2, cpw)
    pltpu.sync_copy(PY2, cph)

    def c2_body(i, _):
        o = i * 16
        ix = idxsv[pl.ds(o, 16)]
        x1s[pl.ds(o, 16)] = plsc.load_gather(cpx1, [ix])
        y1s[pl.ds(o, 16)] = plsc.load_gather(cpy1, [ix])
        x2s[pl.ds(o, 16)] = plsc.load_gather(cpw, [ix])
        y2s[pl.ds(o, 16)] = plsc.load_gather(cph, [ix])
        alivev[pl.ds(o, 16)] = jnp.where(csk[pl.ds(o, 16)] != Y2_INVALID,
                                         jnp.full((16,), 1, jnp.int32),
                                         jnp.zeros((16,), jnp.int32))
        return _

    lax.fori_loop(0, SV, c2_body, 0)
    pltpu.sync_copy(x1s, X1S.at[pl.ds(sbase, ST)])
    pltpu.sync_copy(y1s, Y1S.at[pl.ds(sbase, ST)])
    pltpu.sync_copy(x2s, X2S.at[pl.ds(sbase, ST)])
    pltpu.sync_copy(y2s, Y2S.at[pl.ds(sbase, ST)])

    @pl.when(wid == 0)
    def _():
        for j in range(32):
            keepb[pl.ds(j * 16, 16)] = jnp.zeros((16,), jnp.int32)

    plsc.subcore_barrier()
    pltpu.sync_copy(X1S, fx1)
    pltpu.sync_copy(Y1S, fy1)
    pltpu.sync_copy(X2S, fx2)
    pltpu.sync_copy(Y2S, fy2)
    pltpu.sync_copy(PAR, fden)
    pltpu.sync_copy(PAR.at[pl.ds(sbase, ST)], denv)

    @pl.when(wid == 0)
    def _():
        pltpu.sync_copy(CV.at[pl.ds(0, MP)], fidx)

    def nms_cond(st):
        c, cnt = st
        return (c >= 0) & (cnt < POST)

    def nms_body(st):
        c, cnt = st
        owner = c // 8
        lo = (c % 8) * 16
        co = c * 16

        @pl.when(wid == owner)
        def _():
            av = alivev[pl.ds(lo, 16)]
            x1c = x1s[pl.ds(lo, 16)]
            y1c = y1s[pl.ds(lo, 16)]
            x2c = x2s[pl.ds(lo, 16)]
            y2c = y2s[pl.ds(lo, 16)]
            dc = denv[pl.ds(lo, 16)]
            lanes = _iota16()

            def r_cond(carry):
                av_, km = carry
                return jnp.sum(av_) > 0

            def r_body(carry):
                av_, km = carry
                lrev = plsc.all_reduce_ffs(lax.rev(av_, (0,)) != 0)
                l = 15 - jnp.max(lrev)
                bidx = jnp.zeros((16,), jnp.int32) + (co + l)
                bx1 = plsc.load_gather(fx1, [bidx])
                by1 = plsc.load_gather(fy1, [bidx])
                bx2 = plsc.load_gather(fx2, [bidx])
                xx1 = jnp.maximum(x1c, bx1)
                yy1 = jnp.maximum(y1c, by1)
                xx2 = jnp.minimum(x2c, bx2)
                wd = jnp.maximum(xx2 - xx1 + 1.0, 0.0)
                hd = jnp.maximum(y2c - yy1 + 1.0, 0.0)
                sup = (wd * hd) / dc >= THRESH
                is_l = lanes == l
                av_n = jnp.where(sup | is_l, jnp.zeros((16,), jnp.int32), av_)
                km_n = jnp.where(is_l, jnp.full((16,), 1, jnp.int32), km)
                return av_n, km_n

            av_f, km_f = lax.while_loop(r_cond, r_body,
                                        (av, jnp.zeros((16,), jnp.int32)))
            alivev[pl.ds(lo, 16)] = av_f
            tv16[...] = km_f
            pltpu.sync_copy(tv16, CHMA.at[pl.ds(co, 16)])

        plsc.subcore_barrier()
        pltpu.sync_copy(CHMA.at[pl.ds(co, 16)], tv16)
        km = tv16[...]
        kcnt = jnp.sum(km)

        @pl.when(kcnt > 0)
        def _():
            def ap_cond(mrem):
                return jnp.sum(mrem) > 0

            def ap_body(mrem):
                l0 = jnp.max(plsc.all_reduce_ffs(mrem != 0))
                bidx = jnp.zeros((16,), jnp.int32) + (co + l0)
                bx1 = plsc.load_gather(fx1, [bidx])
                by1 = plsc.load_gather(fy1, [bidx])
                bx2 = plsc.load_gather(fx2, [bidx])

                def s_body(v, _):
                    o = v * 16
                    xx1 = jnp.maximum(x1s[pl.ds(o, 16)], bx1)
                    yy1 = jnp.maximum(y1s[pl.ds(o, 16)], by1)
                    xx2 = jnp.minimum(x2s[pl.ds(o, 16)], bx2)
                    wd = jnp.maximum(xx2 - xx1 + 1.0, 0.0)
                    hd = jnp.maximum(y2s[pl.ds(o, 16)] - yy1 + 1.0, 0.0)
                    sup = (wd * hd) / denv[pl.ds(o, 16)] >= THRESH
                    alivev[pl.ds(o, 16)] = jnp.where(
                        sup, jnp.zeros((16,), jnp.int32), alivev[pl.ds(o, 16)])
                    return _

                lax.fori_loop(0, SV, s_body, 0)
                return jnp.where(_iota16() == l0, jnp.zeros((16,), jnp.int32),
                                 mrem)

            lax.while_loop(ap_cond, ap_body, km)

        @pl.when((wid == 0) & (kcnt > 0))
        def _():
            idxc = fidx[pl.ds(co, 16)]
            rincl = lax.rev(plsc.cumsum(lax.rev(km, (0,))), (0,))
            slot = cnt + rincl - 1
            plsc.store_scatter(keepb, [slot], idxc, mask=km != 0)

        return c - 1, cnt + kcnt

    _, cnt_f = lax.while_loop(nms_cond, nms_body,
                              (jnp.int32(MP // 16 - 1), jnp.int32(0)))

    @pl.when((wid == 0) & (cid == 0))
    def _():
        pltpu.sync_copy(PX1, cpx1)
        pltpu.sync_copy(PY1, cpy1)
        pltpu.sync_copy(PW, cpw)
        pltpu.sync_copy(PH, cph)
        pltpu.sync_copy(PS, cps)

        def e_body(j, _):
            o = j * 16
            jl = o + _iota16()
            kp = keepb[pl.ds(o, 16)]
            valid = (jl < cnt_f) & (jl < POST)
            zf = jnp.zeros((16,), jnp.float32)
            gx1 = jnp.where(valid, plsc.load_gather(cpx1, [kp]), zf)
            gy1 = jnp.where(valid, plsc.load_gather(cpy1, [kp]), zf)
            gw = jnp.where(valid, plsc.load_gather(cpw, [kp]), zf)
            gh = jnp.where(valid, plsc.load_gather(cph, [kp]), zf)
            gs = jnp.where(valid, plsc.load_gather(cps, [kp]), zf)
            plsc.store_scatter(pbuf, [jl * 4], gx1)
            plsc.store_scatter(pbuf, [jl * 4 + 1], gy1)
            plsc.store_scatter(pbuf, [jl * 4 + 2], gw)
            plsc.store_scatter(pbuf, [jl * 4 + 3], gh)
            sbuf[pl.ds(o, 16)] = gs
            return _

        lax.fori_loop(0, 19, e_body, 0)
        pltpu.sync_copy(pbuf.at[pl.ds(0, 1200)], op_hbm)
        pltpu.sync_copy(sbuf, os_hbm)


def _make_sc_call():
    mesh = plsc.VectorSubcoreMesh(core_axis_name="c", subcore_axis_name="s")
    f32 = jnp.float32
    i32 = jnp.int32
    vmem = [
        pltpu.VMEM((PT,), f32),
        pltpu.VMEM((PT * 4,), f32),
        pltpu.VMEM((PT * 4,), f32),
        pltpu.VMEM((PT,), f32),
        pltpu.VMEM((PT,), f32),
        pltpu.VMEM((PT,), f32),
        pltpu.VMEM((PT,), f32),
        pltpu.VMEM((PT,), f32),
        pltpu.VMEM((PT,), f32),
        pltpu.VMEM((PT,), f32),
        pltpu.VMEM((PT,), f32),
        pltpu.VMEM((1280,), i32),
        pltpu.VMEM((1280,), i32),
        pltpu.VMEM((1280,), i32),
        pltpu.VMEM((1280,), i32),
        pltpu.VMEM((PT,), i32),
        pltpu.VMEM((10, 128), i32),
        pltpu.VMEM((256,), i32),
        pltpu.VMEM((4096,), i32),
        pltpu.VMEM((256,), i32),
        pltpu.VMEM((256,), i32),
        pltpu.VMEM((ST,), i32),
        pltpu.VMEM((ST,), i32),
        pltpu.VMEM((ST,), f32),
        pltpu.VMEM((ST,), f32),
        pltpu.VMEM((ST,), f32),
        pltpu.VMEM((ST,), f32),
        pltpu.VMEM((ST,), f32),
        pltpu.VMEM((ST,), f32),
        pltpu.VMEM((ST,), f32),
        pltpu.VMEM((ST,), f32),
        pltpu.VMEM((ST,), f32),
        pltpu.VMEM((ST,), f32),
        pltpu.VMEM((ST,), f32),
        pltpu.VMEM((ST,), f32),
        pltpu.VMEM((ST,), f32),
        pltpu.VMEM((ST,), i32),
        pltpu.VMEM((ST,), i32),
        pltpu.VMEM((MP,), f32),
        pltpu.VMEM((MP,), f32),
        pltpu.VMEM((MP,), f32),
        pltpu.VMEM((MP,), f32),
        pltpu.VMEM((MP,), f32),
        pltpu.VMEM((MP,), f32),
        pltpu.VMEM((MP,), f32),
        pltpu.VMEM((MP,), f32),
        pltpu.VMEM((MP,), f32),
        pltpu.VMEM((MP,), f32),
        pltpu.VMEM((MP,), i32),
        pltpu.VMEM((512,), i32),
        pltpu.VMEM((16,), i32),
        pltpu.VMEM((1216,), f32),
        pltpu.VMEM((304,), f32),
    ]
    shared = [
        pltpu.VMEM_SHARED((MP + 128,), i32),
        pltpu.VMEM_SHARED((MP + 128,), i32),
        pltpu.VMEM_SHARED((MP + 128,), i32),
        pltpu.VMEM_SHARED((MP + 128,), i32),
        pltpu.VMEM_SHARED((NPAD,), f32),
        pltpu.VMEM_SHARED((NPAD,), f32),
        pltpu.VMEM_SHARED((NPAD,), f32),
        pltpu.VMEM_SHARED((NPAD,), f32),
        pltpu.VMEM_SHARED((NPAD,), f32),
        pltpu.VMEM_SHARED((NPAD,), f32),
        pltpu.VMEM_SHARED((NPAD,), f32),
        pltpu.VMEM_SHARED((NPAD,), f32),
        pltpu.VMEM_SHARED((4096,), i32),
        pltpu.VMEM_SHARED((16384,), i32),
        pltpu.VMEM_SHARED((256,), i32),
        pltpu.VMEM_SHARED((MP,), f32),
        pltpu.VMEM_SHARED((MP,), f32),
        pltpu.VMEM_SHARED((MP,), f32),
        pltpu.VMEM_SHARED((MP,), f32),
        pltpu.VMEM_SHARED((MP,), f32),
        pltpu.VMEM_SHARED((MP,), f32),
        pltpu.VMEM_SHARED((MP,), f32),
        pltpu.VMEM_SHARED((MP,), f32),
        pltpu.VMEM_SHARED((MP + 128,), i32),
        pltpu.VMEM_SHARED((MP + 128,), i32),
        pltpu.VMEM_SHARED((MP + 128,), i32),
        pltpu.VMEM_SHARED((MP + 128,), i32),
        pltpu.VMEM_SHARED((MP,), f32),
        pltpu.VMEM_SHARED((MP,), f32),
        pltpu.VMEM_SHARED((MP,), f32),
        pltpu.VMEM_SHARED((MP,), f32),
        pltpu.VMEM_SHARED((MP,), i32),
    ]
    return pl.kernel(
        _sc_body,
        out_type=(jax.ShapeDtypeStruct((1200,), jnp.float32),
                  jax.ShapeDtypeStruct((304,), jnp.float32)),
        mesh=mesh,
        compiler_params=pltpu.CompilerParams(needs_layout_passes=False),
        scratch_types=vmem + shared + [pltpu.SemaphoreType.DMA],
    )


_sc_call = _make_sc_call()


@jax.jit
def kernel(scores, bbox_deltas, image_metadata, boxes):
    del image_metadata
    sc = jnp.reshape(scores, (-1,))
    dl = jnp.reshape(bbox_deltas, (-1,))
    bx = jnp.reshape(boxes, (-1,))
    props_flat, scs_flat = _sc_call(sc, dl, bx)
    proposal_outputs = jnp.reshape(props_flat, (1, POST, 4))
    score_outputs = jnp.reshape(scs_flat[:POST], (1, POST, 1))
    return proposal_outputs, score_outputs


# speedup vs baseline: 1.0277x; 1.0277x over previous
import functools
import jax
import jax.numpy as jnp
import numpy as np
from jax import lax
from jax.experimental import pallas as pl
from jax.experimental.pallas import tpu as pltpu
from jax.experimental.pallas import tpu_sc as plsc

N = 19881
NT = 16
NPAD = 19968
PT = NPAD // NT
PV = PT // 16
M = 2000
MP = 2048
ST = MP // NT
SV = ST // 16
POST = 300
THRESH = 0.7
CLIP = 46.0
KEY_INVALID = np.uint32(0xFFFFFFFF)
Y2_INVALID = np.int32(0x7F800000)
DUMP = NPAD


def _iota16():
    return lax.iota(jnp.int32, 16)


def _extract(vec, lane):
    return jnp.sum(jnp.where(_iota16() == lane, vec, jnp.zeros((16,), vec.dtype)))


def _sc_body(sc_hbm, dl_hbm, bx_hbm, op_hbm, os_hbm,
             in_s, in_d, in_b,
             lx1, ly1, lw, lh, lx2, ly2, lar, lsc, lkey, lval,
             kv, vv, rankv, destv, hist, ha, gbase, tbase,
             csk, csv, px1g, py1g, pwg, phg, px2g, py2g, parg, pscg,
             x1s, y1s, x2s, y2s, denv, alivev, idxsv,
             cpx1, cpy1, cpw, cph, cps,
             fx1, fy1, fx2, fy2, fden, fidx,
             keepb, tv16, pbuf, sbuf,
             CK, CV, NK, NV, X1, Y1, WW, HH, X2, Y2, AR, SS,
             HIST, HISTS, CNTS, PX1, PY1, PW, PH, PX2, PY2, PAR, PS,
             SKA, SVA, SKB, SVB,
             X1S, Y1S, X2S, Y2S, CHMA,
             sem):
    cid = lax.axis_index("c")
    wid = lax.axis_index("s")
    base = wid * PT

    TAIL = N - 15 * PT

    @pl.when(wid < 15)
    def _():
        pltpu.sync_copy(sc_hbm.at[pl.ds(base, PT)], in_s)
        pltpu.sync_copy(dl_hbm.at[pl.ds(base * 4, PT * 4)], in_d)
        pltpu.sync_copy(bx_hbm.at[pl.ds(base * 4, PT * 4)], in_b)

    @pl.when(wid == 15)
    def _():
        b15 = 15 * PT
        pltpu.sync_copy(sc_hbm.at[pl.ds(b15, TAIL)], in_s.at[pl.ds(0, TAIL)])
        pltpu.sync_copy(dl_hbm.at[pl.ds(b15 * 4, TAIL * 4)],
                        in_d.at[pl.ds(0, TAIL * 4)])
        pltpu.sync_copy(bx_hbm.at[pl.ds(b15 * 4, TAIL * 4)],
                        in_b.at[pl.ds(0, TAIL * 4)])

    def a_body(i, _):
        o = i * 16
        lanes = _iota16()
        q = (o + lanes) * 4
        d0 = plsc.load_gather(in_d, [q])
        d1 = plsc.load_gather(in_d, [q + 1])
        d2 = plsc.load_gather(in_d, [q + 2])
        d3 = plsc.load_gather(in_d, [q + 3])
        b0 = plsc.load_gather(in_b, [q])
        b1 = plsc.load_gather(in_b, [q + 1])
        b2 = plsc.load_gather(in_b, [q + 2])
        b3 = plsc.load_gather(in_b, [q + 3])
        s = in_s[pl.ds(o, 16)]
        ax = d0 * b0 + b0
        ay = d1 * b1 + b1
        aw = jnp.exp(d2) * b2
        ah = jnp.exp(d3) * b3
        zero = jnp.float32(0.0)
        ax = jnp.maximum(ax, zero)
        ay = jnp.maximum(ay, zero)
        aw = jnp.maximum(aw, zero)
        ah = jnp.maximum(ah, zero)
        x1 = jnp.minimum(ax, CLIP)
        y1 = jnp.minimum(ay, CLIP)
        x2 = jnp.minimum(ax + aw - 1.0, CLIP)
        y2 = jnp.minimum(ay + ah - 1.0, CLIP)
        w_ = x2 - x1 + 1.0
        h_ = y2 - y1 + 1.0
        e = base + o + lanes
        keep = (w_ >= 2.0) & (h_ >= 2.0) & (e < N)
        u = plsc.bitcast(s + zero, jnp.uint32)
        asc = jnp.where((u >> 31) == jnp.uint32(1), ~u, u | jnp.uint32(0x80000000))
        keyd = jnp.where(keep, ~asc, KEY_INVALID)
        lx1[pl.ds(o, 16)] = x1
        ly1[pl.ds(o, 16)] = y1
        lw[pl.ds(o, 16)] = w_
        lh[pl.ds(o, 16)] = h_
        lx2[pl.ds(o, 16)] = x2
        ly2[pl.ds(o, 16)] = y2
        lar[pl.ds(o, 16)] = w_ * h_
        lsc[pl.ds(o, 16)] = s
        lkey[pl.ds(o, 16)] = plsc.bitcast(keyd, jnp.int32)
        lval[pl.ds(o, 16)] = e
        return _

    lax.fori_loop(0, PV, a_body, 0)
    pltpu.sync_copy(lx1, X1.at[pl.ds(base, PT)])
    pltpu.sync_copy(ly1, Y1.at[pl.ds(base, PT)])
    pltpu.sync_copy(lw, WW.at[pl.ds(base, PT)])
    pltpu.sync_copy(lh, HH.at[pl.ds(base, PT)])
    pltpu.sync_copy(lx2, X2.at[pl.ds(base, PT)])
    pltpu.sync_copy(ly2, Y2.at[pl.ds(base, PT)])
    pltpu.sync_copy(lar, AR.at[pl.ds(base, PT)])
    pltpu.sync_copy(lsc, SS.at[pl.ds(base, PT)])

    P = jnp.uint32(0)
    R = jnp.int32(M)
    for rnd, shift in enumerate((24, 16, 8, 0)):
        dmask = jnp.uint32((0xFFFFFFFF00000000 >> (8 * rnd)) & 0xFFFFFFFF)
        hb = rnd * 4096

        for j in range(16):
            hist[pl.ds(j * 16, 16)] = jnp.zeros((16,), jnp.int32)

        def s_hist(i, _, shift=shift, dmask=dmask, P=P):
            o = i * 16
            ku = plsc.bitcast(lkey[pl.ds(o, 16)], jnp.uint32)
            m = (ku & dmask) == (P & dmask)
            d = ((ku >> jnp.uint32(shift)) & jnp.uint32(255)).astype(jnp.int32)
            cnt, last = plsc.scan_count(d, mask=m)
            plsc.addupdate_scatter(hist, [d], cnt, mask=last)
            return _

        lax.fori_loop(0, PV, s_hist, 0)
        pltpu.sync_copy(hist, HISTS.at[pl.ds(hb + wid * 256, 256)])
        plsc.subcore_barrier()
        pltpu.sync_copy(HISTS.at[pl.ds(hb, 4096)], ha)

        def s_find(j, carry):
            found, bsel, rminus, cumbefore = carry
            o = j * 16
            tot = jnp.zeros((16,), jnp.int32)
            for t in range(16):
                tot = tot + ha[pl.ds(t * 256 + o, 16)]
            incl = plsc.cumsum(tot)
            cum = cumbefore + incl
            l = jnp.max(plsc.all_reduce_ffs(cum >= R))
            this = (l < 16) & jnp.logical_not(found)
            bsel = jnp.where(this, o + l, bsel)
            rminus = jnp.where(this,
                               cumbefore + _extract(incl, l) - _extract(tot, l),
                               rminus)
            return (found | (l < 16), bsel, rminus,
                    cumbefore + jnp.sum(tot))

        _, bsel, rminus, _ = lax.fori_loop(
            0, 16, s_find,
            (jnp.bool_(False), jnp.int32(0), jnp.int32(0), jnp.int32(0)))
        P = P | (bsel.astype(jnp.uint32) << jnp.uint32(shift))
        R = R - rminus
    T = P

    def cnt_body(i, carry):
        nlt, nt = carry
        o = i * 16
        ku = plsc.bitcast(lkey[pl.ds(o, 16)], jnp.uint32)
        one = jnp.full((16,), 1, jnp.int32)
        z = jnp.zeros((16,), jnp.int32)
        return (nlt + jnp.sum(jnp.where(ku < T, one, z)),
                nt + jnp.sum(jnp.where(ku == T, one, z)))

    nlt, ntie = lax.fori_loop(0, PV, cnt_body, (jnp.int32(0), jnp.int32(0)))
    lanes0 = _iota16()
    tv16[...] = (jnp.where(lanes0 == 0, nlt, jnp.zeros((16,), jnp.int32))
                 + jnp.where(lanes0 == 1, ntie, jnp.zeros((16,), jnp.int32)))
    pltpu.sync_copy(tv16, CNTS.at[pl.ds(wid * 16, 16)])
    plsc.subcore_barrier()
    pltpu.sync_copy(CNTS, ha.at[pl.ds(0, 256)])

    def base_body(t, carry):
        blt, btie, c1 = carry
        row = ha[pl.ds(t * 16, 16)]
        nlt_t = _extract(row, 0)
        nt_t = _extract(row, 1)
        zero = jnp.int32(0)
        return (blt + jnp.where(t < wid, nlt_t, zero),
                btie + jnp.where(t < wid, nt_t, zero),
                c1 + nlt_t)

    blt, btie, c1 = lax.fori_loop(0, 16, base_body,
                                  (jnp.int32(0), jnp.int32(0), jnp.int32(0)))

    def dest_rows():
        def d2_body(i, carry):
            lt_run, tie_run = carry
            o = i * 16
            ku = plsc.bitcast(lkey[pl.ds(o, 16)], jnp.uint32)
            mlt = ku < T
            meq = ku == T
            one = jnp.full((16,), 1, jnp.int32)
            z = jnp.zeros((16,), jnp.int32)
            ilt = jnp.where(mlt, one, z)
            ieq = jnp.where(meq, one, z)
            ex_lt = plsc.cumsum(ilt) - ilt
            ex_tie = plsc.cumsum(ieq) - ieq
            d_lt = blt + lt_run + ex_lt
            g = btie + tie_run + ex_tie
            d_tie = c1 + g
            dump = MP + ((o + _iota16()) % 128)
            dest = jnp.where(mlt, d_lt,
                             jnp.where(meq & (g < R), d_tie, dump))
            r_ = i // 8
            destv[r_, pl.ds((i % 8) * 16, 16)] = dest
            return lt_run + jnp.sum(ilt), tie_run + jnp.sum(ieq)

        lax.fori_loop(0, PV, d2_body, (jnp.int32(0), jnp.int32(0)))
        for i in range(PV, 80):
            destv[i // 8, pl.ds((i % 8) * 16, 16)] = MP + (i % 8) * 16 + _iota16()

    dest_rows()
    for r in range(10):
        pltpu.async_copy(lkey.at[pl.ds(r * 128, 128)],
                         SKA.at[destv.at[r]], sem).wait()
        pltpu.async_copy(lval.at[pl.ds(r * 128, 128)],
                         SVA.at[destv.at[r]], sem).wait()

    @pl.when(wid == 0)
    def _():
        for j in range(3):
            tv16[...] = jnp.full((16,), -1, jnp.int32)
            pltpu.sync_copy(tv16, SKA.at[pl.ds(M + j * 16, 16)])
            tv16[...] = jnp.zeros((16,), jnp.int32)
            pltpu.sync_copy(tv16, SVA.at[pl.ds(M + j * 16, 16)])

    plsc.subcore_barrier()

    def radix_pass(src_k, src_v, dst_k, dst_v, shift, nv, per_tile, nrows):
        b0 = wid * per_tile
        pltpu.sync_copy(src_k.at[pl.ds(b0, per_tile)], kv.at[pl.ds(0, per_tile)])
        pltpu.sync_copy(src_v.at[pl.ds(b0, per_tile)], vv.at[pl.ds(0, per_tile)])
        for j in range(16):
            hist[pl.ds(j * 16, 16)] = jnp.zeros((16,), jnp.int32)

        def h_body(i, _):
            o = i * 16
            ku = plsc.bitcast(kv[pl.ds(o, 16)], jnp.uint32)
            d = ((ku >> shift) & jnp.uint32(255)).astype(jnp.int32)
            pre = plsc.load_gather(hist, [d])
            cnt, last = plsc.scan_count(d)
            rankv[pl.ds(o, 16)] = pre + cnt - 1
            plsc.addupdate_scatter(hist, [d], cnt, mask=last)
            return _

        lax.fori_loop(0, nv, h_body, 0)
        pltpu.sync_copy(hist, HIST.at[pl.ds(wid * 256, 256)])
        plsc.subcore_barrier()
        pltpu.sync_copy(HIST, ha)

        def g_body(j, carry):
            o = j * 16
            tot = jnp.zeros((16,), jnp.int32)
            tb = jnp.zeros((16,), jnp.int32)
            for t in range(16):
                row = ha[pl.ds(t * 256 + o, 16)]
                tot = tot + row
                tb = tb + jnp.where(jnp.int32(t) < wid, row,
                                    jnp.zeros((16,), jnp.int32))
            incl = plsc.cumsum(tot)
            gbase[pl.ds(o, 16)] = incl - tot + carry
            tbase[pl.ds(o, 16)] = tb
            return carry + jnp.sum(tot)

        lax.fori_loop(0, 16, g_body, jnp.int32(0))

        for r in range(nrows):
            vlo = r * 8
            vhi = min(r * 8 + 8, nv)

            def d_body(i, _, vlo=vlo, r=r):
                o = i * 16
                ku = plsc.bitcast(kv[pl.ds(vlo * 16 + o, 16)], jnp.uint32)
                d = ((ku >> shift) & jnp.uint32(255)).astype(jnp.int32)
                gb = plsc.load_gather(gbase, [d])
                tb = plsc.load_gather(tbase, [d])
                dst = gb + tb + rankv[pl.ds(vlo * 16 + o, 16)]
                destv[r, pl.ds(o, 16)] = dst
                return _

            lax.fori_loop(0, vhi - vlo, d_body, 0)
            for i in range(vhi - vlo, 8):
                destv[r, pl.ds(i * 16, 16)] = DUMP + i * 16 + _iota16()
        for r in range(nrows):
            pltpu.async_copy(kv.at[pl.ds(r * 128, 128)],
                             dst_k.at[destv.at[r]], sem).wait()
            pltpu.async_copy(vv.at[pl.ds(r * 128, 128)],
                             dst_v.at[destv.at[r]], sem).wait()
        plsc.subcore_barrier()

    radix_pass(SKA, SVA, SKB, SVB, jnp.uint32(0), SV, ST, 1)
    radix_pass(SKB, SVB, SKA, SVA, jnp.uint32(8), SV, ST, 1)
    radix_pass(SKA, SVA, SKB, SVB, jnp.uint32(16), SV, ST, 1)
    radix_pass(SKB, SVB, SKA, SVA, jnp.uint32(24), SV, ST, 1)

    sbase = wid * ST
    pltpu.sync_copy(SKA.at[pl.ds(sbase, ST)], csk)
    pltpu.sync_copy(SVA.at[pl.ds(sbase, ST)], csv)
    g1 = pltpu.make_async_copy(X1.at[csv], px1g, sem)
    g2 = pltpu.make_async_copy(Y1.at[csv], py1g, sem)
    g3 = pltpu.make_async_copy(WW.at[csv], pwg, sem)
    g4 = pltpu.make_async_copy(HH.at[csv], phg, sem)
    g5 = pltpu.make_async_copy(X2.at[csv], px2g, sem)
    g6 = pltpu.make_async_copy(Y2.at[csv], py2g, sem)
    g7 = pltpu.make_async_copy(AR.at[csv], parg, sem)
    g8 = pltpu.make_async_copy(SS.at[csv], pscg, sem)
    for g in (g1, g2, g3, g4, g5, g6, g7, g8):
        g.start()
    for g in (g1, g2, g3, g4, g5, g6, g7, g8):
        g.wait()
    pltpu.sync_copy(px1g, PX1.at[pl.ds(sbase, ST)])
    pltpu.sync_copy(py1g, PY1.at[pl.ds(sbase, ST)])
    pltpu.sync_copy(pwg, PW.at[pl.ds(sbase, ST)])
    pltpu.sync_copy(phg, PH.at[pl.ds(sbase, ST)])
    pltpu.sync_copy(px2g, PX2.at[pl.ds(sbase, ST)])
    pltpu.sync_copy(py2g, PY2.at[pl.ds(sbase, ST)])
    pltpu.sync_copy(parg, PAR.at[pl.ds(sbase, ST)])
    pltpu.sync_copy(pscg, PS.at[pl.ds(sbase, ST)])

    def y_body(i, _):
        o = i * 16
        pos = sbase + o + _iota16()
        kk = plsc.bitcast(csk[pl.ds(o, 16)], jnp.uint32)
        valid = (kk != KEY_INVALID) & (pos < M)
        y2b = plsc.bitcast(py2g[pl.ds(o, 16)], jnp.int32)
        kv[pl.ds(o, 16)] = jnp.where(valid, y2b,
                                     jnp.full((16,), Y2_INVALID, jnp.int32))
        vv[pl.ds(o, 16)] = pos
        return _

    lax.fori_loop(0, SV, y_body, 0)
    pltpu.sync_copy(kv.at[pl.ds(0, ST)], CK.at[pl.ds(sbase, ST)])
    pltpu.sync_copy(vv.at[pl.ds(0, ST)], CV.at[pl.ds(sbase, ST)])
    plsc.subcore_barrier()

    radix_pass(CK, CV, NK, NV, jnp.uint32(0), SV, ST, 1)
    radix_pass(NK, NV, CK, CV, jnp.uint32(8), SV, ST, 1)
    radix_pass(CK, CV, NK, NV, jnp.uint32(16), SV, ST, 1)
    radix_pass(NK, NV, CK, CV, jnp.uint32(24), SV, ST, 1)

    pltpu.sync_copy(CK.at[pl.ds(sbase, ST)], csk)
    pltpu.sync_copy(CV.at[pl.ds(sbase, ST)], idxsv)
    pltpu.sync_copy(PX1, cpx1)
    pltpu.sync_copy(PY1, cpy1)
    pltpu.sync_copy(PX2, cpw)
    pltpu.sync_copy(PY2, cph)

    def c2_body(i, _):
        o = i * 16
        ix = idxsv[pl.ds(o, 16)]
        x1s[pl.ds(o, 16)] = plsc.load_gather(cpx1, [ix])
        y1s[pl.ds(o, 16)] = plsc.load_gather(cpy1, [ix])
        x2s[pl.ds(o, 16)] = plsc.load_gather(cpw, [ix])
        y2s[pl.ds(o, 16)] = plsc.load_gather(cph, [ix])
        alivev[pl.ds(o, 16)] = jnp.where(csk[pl.ds(o, 16)] != Y2_INVALID,
                                         jnp.full((16,), 1, jnp.int32),
                                         jnp.zeros((16,), jnp.int32))
        return _

    lax.fori_loop(0, SV, c2_body, 0)
    pltpu.sync_copy(x1s, X1S.at[pl.ds(sbase, ST)])
    pltpu.sync_copy(y1s, Y1S.at[pl.ds(sbase, ST)])
    pltpu.sync_copy(x2s, X2S.at[pl.ds(sbase, ST)])
    pltpu.sync_copy(y2s, Y2S.at[pl.ds(sbase, ST)])

    @pl.when(wid == 0)
    def _():
        for j in range(32):
            keepb[pl.ds(j * 16, 16)] = jnp.zeros((16,), jnp.int32)

    plsc.subcore_barrier()
    pltpu.sync_copy(X1S, fx1)
    pltpu.sync_copy(Y1S, fy1)
    pltpu.sync_copy(X2S, fx2)
    pltpu.sync_copy(Y2S, fy2)
    pltpu.sync_copy(PAR, fden)
    pltpu.sync_copy(PAR.at[pl.ds(sbase, ST)], denv)

    @pl.when(wid == 0)
    def _():
        pltpu.sync_copy(CV.at[pl.ds(0, MP)], fidx)

    def nms_cond(st):
        c, cnt = st
        return (c >= 0) & (cnt < POST)

    def nms_body(st):
        c, cnt = st
        owner = c // 8
        lo = (c % 8) * 16
        co = c * 16

        @pl.when(wid == owner)
        def _():
            av = alivev[pl.ds(lo, 16)]
            x1c = x1s[pl.ds(lo, 16)]
            y1c = y1s[pl.ds(lo, 16)]
            x2c = x2s[pl.ds(lo, 16)]
            y2c = y2s[pl.ds(lo, 16)]
            dc = denv[pl.ds(lo, 16)]
            lanes = _iota16()

            def r_body(l_, carry):
                av_, km = carry
                l = 15 - l_
                al = _extract(av_, l) > 0
                bidx = jnp.zeros((16,), jnp.int32) + (co + l)
                bx1 = plsc.load_gather(fx1, [bidx])
                by1 = plsc.load_gather(fy1, [bidx])
                bx2 = plsc.load_gather(fx2, [bidx])
                xx1 = jnp.maximum(x1c, bx1)
                yy1 = jnp.maximum(y1c, by1)
                xx2 = jnp.minimum(x2c, bx2)
                wd = jnp.maximum(xx2 - xx1 + 1.0, 0.0)
                hd = jnp.maximum(y2c - yy1 + 1.0, 0.0)
                sup = (wd * hd) / dc >= THRESH
                is_l = lanes == l
                nav = jnp.where(sup | is_l, jnp.zeros((16,), jnp.int32), av_)
                av_n = jnp.where(al, nav, av_)
                km_n = jnp.where(al & is_l, jnp.full((16,), 1, jnp.int32), km)
                return av_n, km_n

            av_f, km_f = lax.fori_loop(0, 16, r_body,
                                       (av, jnp.zeros((16,), jnp.int32)))
            alivev[pl.ds(lo, 16)] = av_f
            tv16[...] = km_f
            pltpu.sync_copy(tv16, CHMA.at[pl.ds(co, 16)])

        plsc.subcore_barrier()
        pltpu.sync_copy(CHMA.at[pl.ds(co, 16)], tv16)
        km = tv16[...]
        kcnt = jnp.sum(km)

        @pl.when(kcnt > 0)
        def _():
            def ap_cond(mrem):
                return jnp.sum(mrem) > 0

            def ap_body(mrem):
                l0 = jnp.max(plsc.all_reduce_ffs(mrem != 0))
                bidx = jnp.zeros((16,), jnp.int32) + (co + l0)
                bx1 = plsc.load_gather(fx1, [bidx])
                by1 = plsc.load_gather(fy1, [bidx])
                bx2 = plsc.load_gather(fx2, [bidx])

                def s_body(v, _):
                    o = v * 16
                    xx1 = jnp.maximum(x1s[pl.ds(o, 16)], bx1)
                    yy1 = jnp.maximum(y1s[pl.ds(o, 16)], by1)
                    xx2 = jnp.minimum(x2s[pl.ds(o, 16)], bx2)
                    wd = jnp.maximum(xx2 - xx1 + 1.0, 0.0)
                    hd = jnp.maximum(y2s[pl.ds(o, 16)] - yy1 + 1.0, 0.0)
                    sup = (wd * hd) / denv[pl.ds(o, 16)] >= THRESH
                    alivev[pl.ds(o, 16)] = jnp.where(
                        sup, jnp.zeros((16,), jnp.int32), alivev[pl.ds(o, 16)])
                    return _

                lax.fori_loop(0, SV, s_body, 0)
                return jnp.where(_iota16() == l0, jnp.zeros((16,), jnp.int32),
                                 mrem)

            lax.while_loop(ap_cond, ap_body, km)

        @pl.when((wid == 0) & (kcnt > 0))
        def _():
            idxc = fidx[pl.ds(co, 16)]
            rincl = lax.rev(plsc.cumsum(lax.rev(km, (0,))), (0,))
            slot = cnt + rincl - 1
            plsc.store_scatter(keepb, [slot], idxc, mask=km != 0)

        return c - 1, cnt + kcnt

    _, cnt_f = lax.while_loop(nms_cond, nms_body,
                              (jnp.int32(MP // 16 - 1), jnp.int32(0)))

    @pl.when((wid == 0) & (cid == 0))
    def _():
        pltpu.sync_copy(PX1, cpx1)
        pltpu.sync_copy(PY1, cpy1)
        pltpu.sync_copy(PW, cpw)
        pltpu.sync_copy(PH, cph)
        pltpu.sync_copy(PS, cps)

        def e_body(j, _):
            o = j * 16
            jl = o + _iota16()
            kp = keepb[pl.ds(o, 16)]
            valid = (jl < cnt_f) & (jl < POST)
            zf = jnp.zeros((16,), jnp.float32)
            gx1 = jnp.where(valid, plsc.load_gather(cpx1, [kp]), zf)
            gy1 = jnp.where(valid, plsc.load_gather(cpy1, [kp]), zf)
            gw = jnp.where(valid, plsc.load_gather(cpw, [kp]), zf)
            gh = jnp.where(valid, plsc.load_gather(cph, [kp]), zf)
            gs = jnp.where(valid, plsc.load_gather(cps, [kp]), zf)
            plsc.store_scatter(pbuf, [jl * 4], gx1)
            plsc.store_scatter(pbuf, [jl * 4 + 1], gy1)
            plsc.store_scatter(pbuf, [jl * 4 + 2], gw)
            plsc.store_scatter(pbuf, [jl * 4 + 3], gh)
            sbuf[pl.ds(o, 16)] = gs
            return _

        lax.fori_loop(0, 19, e_body, 0)
        pltpu.sync_copy(pbuf.at[pl.ds(0, 1200)], op_hbm)
        pltpu.sync_copy(sbuf, os_hbm)


def _make_sc_call():
    mesh = plsc.VectorSubcoreMesh(core_axis_name="c", subcore_axis_name="s")
    f32 = jnp.float32
    i32 = jnp.int32
    vmem = [
        pltpu.VMEM((PT,), f32),
        pltpu.VMEM((PT * 4,), f32),
        pltpu.VMEM((PT * 4,), f32),
        pltpu.VMEM((PT,), f32),
        pltpu.VMEM((PT,), f32),
        pltpu.VMEM((PT,), f32),
        pltpu.VMEM((PT,), f32),
        pltpu.VMEM((PT,), f32),
        pltpu.VMEM((PT,), f32),
        pltpu.VMEM((PT,), f32),
        pltpu.VMEM((PT,), f32),
        pltpu.VMEM((1280,), i32),
        pltpu.VMEM((1280,), i32),
        pltpu.VMEM((1280,), i32),
        pltpu.VMEM((1280,), i32),
        pltpu.VMEM((PT,), i32),
        pltpu.VMEM((10, 128), i32),
        pltpu.VMEM((256,), i32),
        pltpu.VMEM((4096,), i32),
        pltpu.VMEM((256,), i32),
        pltpu.VMEM((256,), i32),
        pltpu.VMEM((ST,), i32),
        pltpu.VMEM((ST,), i32),
        pltpu.VMEM((ST,), f32),
        pltpu.VMEM((ST,), f32),
        pltpu.VMEM((ST,), f32),
        pltpu.VMEM((ST,), f32),
        pltpu.VMEM((ST,), f32),
        pltpu.VMEM((ST,), f32),
        pltpu.VMEM((ST,), f32),
        pltpu.VMEM((ST,), f32),
        pltpu.VMEM((ST,), f32),
        pltpu.VMEM((ST,), f32),
        pltpu.VMEM((ST,), f32),
        pltpu.VMEM((ST,), f32),
        pltpu.VMEM((ST,), f32),
        pltpu.VMEM((ST,), i32),
        pltpu.VMEM((ST,), i32),
        pltpu.VMEM((MP,), f32),
        pltpu.VMEM((MP,), f32),
        pltpu.VMEM((MP,), f32),
        pltpu.VMEM((MP,), f32),
        pltpu.VMEM((MP,), f32),
        pltpu.VMEM((MP,), f32),
        pltpu.VMEM((MP,), f32),
        pltpu.VMEM((MP,), f32),
        pltpu.VMEM((MP,), f32),
        pltpu.VMEM((MP,), f32),
        pltpu.VMEM((MP,), i32),
        pltpu.VMEM((512,), i32),
        pltpu.VMEM((16,), i32),
        pltpu.VMEM((1216,), f32),
        pltpu.VMEM((304,), f32),
    ]
    shared = [
        pltpu.VMEM_SHARED((MP + 128,), i32),
        pltpu.VMEM_SHARED((MP + 128,), i32),
        pltpu.VMEM_SHARED((MP + 128,), i32),
        pltpu.VMEM_SHARED((MP + 128,), i32),
        pltpu.VMEM_SHARED((NPAD,), f32),
        pltpu.VMEM_SHARED((NPAD,), f32),
        pltpu.VMEM_SHARED((NPAD,), f32),
        pltpu.VMEM_SHARED((NPAD,), f32),
        pltpu.VMEM_SHARED((NPAD,), f32),
        pltpu.VMEM_SHARED((NPAD,), f32),
        pltpu.VMEM_SHARED((NPAD,), f32),
        pltpu.VMEM_SHARED((NPAD,), f32),
        pltpu.VMEM_SHARED((4096,), i32),
        pltpu.VMEM_SHARED((16384,), i32),
        pltpu.VMEM_SHARED((256,), i32),
        pltpu.VMEM_SHARED((MP,), f32),
        pltpu.VMEM_SHARED((MP,), f32),
        pltpu.VMEM_SHARED((MP,), f32),
        pltpu.VMEM_SHARED((MP,), f32),
        pltpu.VMEM_SHARED((MP,), f32),
        pltpu.VMEM_SHARED((MP,), f32),
        pltpu.VMEM_SHARED((MP,), f32),
        pltpu.VMEM_SHARED((MP,), f32),
        pltpu.VMEM_SHARED((MP + 128,), i32),
        pltpu.VMEM_SHARED((MP + 128,), i32),
        pltpu.VMEM_SHARED((MP + 128,), i32),
        pltpu.VMEM_SHARED((MP + 128,), i32),
        pltpu.VMEM_SHARED((MP,), f32),
        pltpu.VMEM_SHARED((MP,), f32),
        pltpu.VMEM_SHARED((MP,), f32),
        pltpu.VMEM_SHARED((MP,), f32),
        pltpu.VMEM_SHARED((MP,), i32),
    ]
    return pl.kernel(
        _sc_body,
        out_type=(jax.ShapeDtypeStruct((1200,), jnp.float32),
                  jax.ShapeDtypeStruct((304,), jnp.float32)),
        mesh=mesh,
        compiler_params=pltpu.CompilerParams(needs_layout_passes=False),
        scratch_types=vmem + shared + [pltpu.SemaphoreType.DMA],
    )


_sc_call = _make_sc_call()


@jax.jit
def kernel(scores, bbox_deltas, image_metadata, boxes):
    del image_metadata
    sc = jnp.reshape(scores, (-1,))
    dl = jnp.reshape(bbox_deltas, (-1,))
    bx = jnp.reshape(boxes, (-1,))
    props_flat, scs_flat = _sc_call(sc, dl, bx)
    proposal_outputs = jnp.reshape(props_flat, (1, POST, 4))
    score_outputs = jnp.reshape(scs_flat[:POST], (1, POST, 1))
    return proposal_outputs, score_outputs


# speedup vs baseline: 1.0712x; 1.0423x over previous
import functools
import jax
import jax.numpy as jnp
import numpy as np
from jax import lax
from jax.experimental import pallas as pl
from jax.experimental.pallas import tpu as pltpu
from jax.experimental.pallas import tpu_sc as plsc

N = 19881
NT = 16
NPAD = 19968
PT = NPAD // NT
PV = PT // 16
M = 2000
MP = 2048
ST = MP // NT
SV = ST // 16
POST = 300
THRESH = 0.7
CLIP = 46.0
KEY_INVALID = np.uint32(0xFFFFFFFF)
Y2_INVALID = np.int32(0x7F800000)
DUMP = NPAD


def _iota16():
    return lax.iota(jnp.int32, 16)


def _extract(vec, lane):
    return jnp.sum(jnp.where(_iota16() == lane, vec, jnp.zeros((16,), vec.dtype)))


def _sc_body(sc_hbm, dl_hbm, bx_hbm, op_hbm, os_hbm,
             in_s, in_d, in_b,
             lx1, ly1, lw, lh, lx2, ly2, lar, lsc, lkey, lval,
             kv, vv, rankv, destv, hist, ha, gbase, tbase,
             csk, csv, px1g, py1g, pwg, phg, px2g, py2g, parg, pscg,
             x1s, y1s, x2s, y2s, denv, alivev, idxsv,
             cpx1, cpy1, cpw, cph, cps,
             fx1, fy1, fx2, fy2, fden, fidx,
             keepb, tv16, pbuf, sbuf,
             CK, CV, NK, NV, X1, Y1, WW, HH, X2, Y2, AR, SS,
             HIST, HISTS, CNTS, PX1, PY1, PW, PH, PX2, PY2, PAR, PS,
             SKA, SVA, SKB, SVB,
             X1S, Y1S, X2S, Y2S, CHMA,
             sem):
    cid = lax.axis_index("c")
    wid = lax.axis_index("s")
    base = wid * PT

    TAIL = N - 15 * PT

    @pl.when(wid < 15)
    def _():
        ins = [
            pltpu.make_async_copy(sc_hbm.at[pl.ds(base, PT)], in_s, sem),
            pltpu.make_async_copy(dl_hbm.at[pl.ds(base * 4, PT * 4)], in_d, sem),
            pltpu.make_async_copy(bx_hbm.at[pl.ds(base * 4, PT * 4)], in_b, sem),
        ]
        for g in ins:
            g.start()
        for g in ins:
            g.wait()

    @pl.when(wid == 15)
    def _():
        b15 = 15 * PT
        ins = [
            pltpu.make_async_copy(sc_hbm.at[pl.ds(b15, TAIL)],
                                  in_s.at[pl.ds(0, TAIL)], sem),
            pltpu.make_async_copy(dl_hbm.at[pl.ds(b15 * 4, TAIL * 4)],
                                  in_d.at[pl.ds(0, TAIL * 4)], sem),
            pltpu.make_async_copy(bx_hbm.at[pl.ds(b15 * 4, TAIL * 4)],
                                  in_b.at[pl.ds(0, TAIL * 4)], sem),
        ]
        for g in ins:
            g.start()
        for g in ins:
            g.wait()

    def a_body(i, _):
        o = i * 16
        lanes = _iota16()
        q = (o + lanes) * 4
        d0 = plsc.load_gather(in_d, [q])
        d1 = plsc.load_gather(in_d, [q + 1])
        d2 = plsc.load_gather(in_d, [q + 2])
        d3 = plsc.load_gather(in_d, [q + 3])
        b0 = plsc.load_gather(in_b, [q])
        b1 = plsc.load_gather(in_b, [q + 1])
        b2 = plsc.load_gather(in_b, [q + 2])
        b3 = plsc.load_gather(in_b, [q + 3])
        s = in_s[pl.ds(o, 16)]
        ax = d0 * b0 + b0
        ay = d1 * b1 + b1
        aw = jnp.exp(d2) * b2
        ah = jnp.exp(d3) * b3
        zero = jnp.float32(0.0)
        ax = jnp.maximum(ax, zero)
        ay = jnp.maximum(ay, zero)
        aw = jnp.maximum(aw, zero)
        ah = jnp.maximum(ah, zero)
        x1 = jnp.minimum(ax, CLIP)
        y1 = jnp.minimum(ay, CLIP)
        x2 = jnp.minimum(ax + aw - 1.0, CLIP)
        y2 = jnp.minimum(ay + ah - 1.0, CLIP)
        w_ = x2 - x1 + 1.0
        h_ = y2 - y1 + 1.0
        e = base + o + lanes
        keep = (w_ >= 2.0) & (h_ >= 2.0) & (e < N)
        u = plsc.bitcast(s + zero, jnp.uint32)
        asc = jnp.where((u >> 31) == jnp.uint32(1), ~u, u | jnp.uint32(0x80000000))
        keyd = jnp.where(keep, ~asc, KEY_INVALID)
        lx1[pl.ds(o, 16)] = x1
        ly1[pl.ds(o, 16)] = y1
        lw[pl.ds(o, 16)] = w_
        lh[pl.ds(o, 16)] = h_
        lx2[pl.ds(o, 16)] = x2
        ly2[pl.ds(o, 16)] = y2
        lar[pl.ds(o, 16)] = w_ * h_
        lsc[pl.ds(o, 16)] = s
        lkey[pl.ds(o, 16)] = plsc.bitcast(keyd, jnp.int32)
        lval[pl.ds(o, 16)] = e
        return _

    lax.fori_loop(0, PV, a_body, 0)
    a_outs = [
        pltpu.make_async_copy(lx1, X1.at[pl.ds(base, PT)], sem),
        pltpu.make_async_copy(ly1, Y1.at[pl.ds(base, PT)], sem),
        pltpu.make_async_copy(lw, WW.at[pl.ds(base, PT)], sem),
        pltpu.make_async_copy(lh, HH.at[pl.ds(base, PT)], sem),
        pltpu.make_async_copy(lx2, X2.at[pl.ds(base, PT)], sem),
        pltpu.make_async_copy(ly2, Y2.at[pl.ds(base, PT)], sem),
        pltpu.make_async_copy(lar, AR.at[pl.ds(base, PT)], sem),
        pltpu.make_async_copy(lsc, SS.at[pl.ds(base, PT)], sem),
    ]
    for g in a_outs:
        g.start()
    for g in a_outs:
        g.wait()

    P = jnp.uint32(0)
    R = jnp.int32(M)
    for rnd, shift in enumerate((24, 16, 8, 0)):
        dmask = jnp.uint32((0xFFFFFFFF00000000 >> (8 * rnd)) & 0xFFFFFFFF)
        hb = rnd * 4096

        for j in range(16):
            hist[pl.ds(j * 16, 16)] = jnp.zeros((16,), jnp.int32)

        def s_hist(i, _, shift=shift, dmask=dmask, P=P):
            o = i * 16
            ku = plsc.bitcast(lkey[pl.ds(o, 16)], jnp.uint32)
            m = (ku & dmask) == (P & dmask)
            d = ((ku >> jnp.uint32(shift)) & jnp.uint32(255)).astype(jnp.int32)
            cnt, last = plsc.scan_count(d, mask=m)
            plsc.addupdate_scatter(hist, [d], cnt, mask=last)
            return _

        lax.fori_loop(0, PV, s_hist, 0)
        pltpu.sync_copy(hist, HISTS.at[pl.ds(hb + wid * 256, 256)])
        plsc.subcore_barrier()
        pltpu.sync_copy(HISTS.at[pl.ds(hb, 4096)], ha)

        def s_find(j, carry):
            found, bsel, rminus, cumbefore = carry
            o = j * 16
            tot = jnp.zeros((16,), jnp.int32)
            for t in range(16):
                tot = tot + ha[pl.ds(t * 256 + o, 16)]
            incl = plsc.cumsum(tot)
            cum = cumbefore + incl
            l = jnp.max(plsc.all_reduce_ffs(cum >= R))
            this = (l < 16) & jnp.logical_not(found)
            bsel = jnp.where(this, o + l, bsel)
            rminus = jnp.where(this,
                               cumbefore + _extract(incl, l) - _extract(tot, l),
                               rminus)
            return (found | (l < 16), bsel, rminus,
                    cumbefore + jnp.sum(tot))

        _, bsel, rminus, _ = lax.fori_loop(
            0, 16, s_find,
            (jnp.bool_(False), jnp.int32(0), jnp.int32(0), jnp.int32(0)))
        P = P | (bsel.astype(jnp.uint32) << jnp.uint32(shift))
        R = R - rminus
    T = P

    def cnt_body(i, carry):
        nlt, nt = carry
        o = i * 16
        ku = plsc.bitcast(lkey[pl.ds(o, 16)], jnp.uint32)
        one = jnp.full((16,), 1, jnp.int32)
        z = jnp.zeros((16,), jnp.int32)
        return (nlt + jnp.sum(jnp.where(ku < T, one, z)),
                nt + jnp.sum(jnp.where(ku == T, one, z)))

    nlt, ntie = lax.fori_loop(0, PV, cnt_body, (jnp.int32(0), jnp.int32(0)))
    lanes0 = _iota16()
    tv16[...] = (jnp.where(lanes0 == 0, nlt, jnp.zeros((16,), jnp.int32))
                 + jnp.where(lanes0 == 1, ntie, jnp.zeros((16,), jnp.int32)))
    pltpu.sync_copy(tv16, CNTS.at[pl.ds(wid * 16, 16)])
    plsc.subcore_barrier()
    pltpu.sync_copy(CNTS, ha.at[pl.ds(0, 256)])

    def base_body(t, carry):
        blt, btie, c1 = carry
        row = ha[pl.ds(t * 16, 16)]
        nlt_t = _extract(row, 0)
        nt_t = _extract(row, 1)
        zero = jnp.int32(0)
        return (blt + jnp.where(t < wid, nlt_t, zero),
                btie + jnp.where(t < wid, nt_t, zero),
                c1 + nlt_t)

    blt, btie, c1 = lax.fori_loop(0, 16, base_body,
                                  (jnp.int32(0), jnp.int32(0), jnp.int32(0)))

    def dest_rows():
        def d2_body(i, carry):
            lt_run, tie_run = carry
            o = i * 16
            ku = plsc.bitcast(lkey[pl.ds(o, 16)], jnp.uint32)
            mlt = ku < T
            meq = ku == T
            one = jnp.full((16,), 1, jnp.int32)
            z = jnp.zeros((16,), jnp.int32)
            ilt = jnp.where(mlt, one, z)
            ieq = jnp.where(meq, one, z)
            ex_lt = plsc.cumsum(ilt) - ilt
            ex_tie = plsc.cumsum(ieq) - ieq
            d_lt = blt + lt_run + ex_lt
            g = btie + tie_run + ex_tie
            d_tie = c1 + g
            dump = MP + ((o + _iota16()) % 128)
            dest = jnp.where(mlt, d_lt,
                             jnp.where(meq & (g < R), d_tie, dump))
            r_ = i // 8
            destv[r_, pl.ds((i % 8) * 16, 16)] = dest
            return lt_run + jnp.sum(ilt), tie_run + jnp.sum(ieq)

        lax.fori_loop(0, PV, d2_body, (jnp.int32(0), jnp.int32(0)))
        for i in range(PV, 80):
            destv[i // 8, pl.ds((i % 8) * 16, 16)] = MP + (i % 8) * 16 + _iota16()

    dest_rows()
    s2_dmas = []
    for r in range(10):
        s2_dmas.append(pltpu.make_async_copy(lkey.at[pl.ds(r * 128, 128)],
                                             SKA.at[destv.at[r]], sem))
        s2_dmas.append(pltpu.make_async_copy(lval.at[pl.ds(r * 128, 128)],
                                             SVA.at[destv.at[r]], sem))
    for g in s2_dmas:
        g.start()
    for g in s2_dmas:
        g.wait()

    @pl.when(wid == 0)
    def _():
        for j in range(3):
            tv16[...] = jnp.full((16,), -1, jnp.int32)
            pltpu.sync_copy(tv16, SKA.at[pl.ds(M + j * 16, 16)])
            tv16[...] = jnp.zeros((16,), jnp.int32)
            pltpu.sync_copy(tv16, SVA.at[pl.ds(M + j * 16, 16)])

    plsc.subcore_barrier()

    def radix_pass(src_k, src_v, dst_k, dst_v, shift, nv, per_tile, nrows):
        b0 = wid * per_tile
        rp_ins = [
            pltpu.make_async_copy(src_k.at[pl.ds(b0, per_tile)],
                                  kv.at[pl.ds(0, per_tile)], sem),
            pltpu.make_async_copy(src_v.at[pl.ds(b0, per_tile)],
                                  vv.at[pl.ds(0, per_tile)], sem),
        ]
        for g in rp_ins:
            g.start()
        for g in rp_ins:
            g.wait()
        for j in range(16):
            hist[pl.ds(j * 16, 16)] = jnp.zeros((16,), jnp.int32)

        def h_body(i, _):
            o = i * 16
            ku = plsc.bitcast(kv[pl.ds(o, 16)], jnp.uint32)
            d = ((ku >> shift) & jnp.uint32(255)).astype(jnp.int32)
            pre = plsc.load_gather(hist, [d])
            cnt, last = plsc.scan_count(d)
            rankv[pl.ds(o, 16)] = pre + cnt - 1
            plsc.addupdate_scatter(hist, [d], cnt, mask=last)
            return _

        lax.fori_loop(0, nv, h_body, 0)
        pltpu.sync_copy(hist, HIST.at[pl.ds(wid * 256, 256)])
        plsc.subcore_barrier()
        pltpu.sync_copy(HIST, ha)

        def g_body(j, carry):
            o = j * 16
            tot = jnp.zeros((16,), jnp.int32)
            tb = jnp.zeros((16,), jnp.int32)
            for t in range(16):
                row = ha[pl.ds(t * 256 + o, 16)]
                tot = tot + row
                tb = tb + jnp.where(jnp.int32(t) < wid, row,
                                    jnp.zeros((16,), jnp.int32))
            incl = plsc.cumsum(tot)
            gbase[pl.ds(o, 16)] = incl - tot + carry
            tbase[pl.ds(o, 16)] = tb
            return carry + jnp.sum(tot)

        lax.fori_loop(0, 16, g_body, jnp.int32(0))

        for r in range(nrows):
            vlo = r * 8
            vhi = min(r * 8 + 8, nv)

            def d_body(i, _, vlo=vlo, r=r):
                o = i * 16
                ku = plsc.bitcast(kv[pl.ds(vlo * 16 + o, 16)], jnp.uint32)
                d = ((ku >> shift) & jnp.uint32(255)).astype(jnp.int32)
                gb = plsc.load_gather(gbase, [d])
                tb = plsc.load_gather(tbase, [d])
                dst = gb + tb + rankv[pl.ds(vlo * 16 + o, 16)]
                destv[r, pl.ds(o, 16)] = dst
                return _

            lax.fori_loop(0, vhi - vlo, d_body, 0)
            for i in range(vhi - vlo, 8):
                destv[r, pl.ds(i * 16, 16)] = DUMP + i * 16 + _iota16()
        rp_outs = []
        for r in range(nrows):
            rp_outs.append(pltpu.make_async_copy(kv.at[pl.ds(r * 128, 128)],
                                                 dst_k.at[destv.at[r]], sem))
            rp_outs.append(pltpu.make_async_copy(vv.at[pl.ds(r * 128, 128)],
                                                 dst_v.at[destv.at[r]], sem))
        for g in rp_outs:
            g.start()
        for g in rp_outs:
            g.wait()
        plsc.subcore_barrier()

    radix_pass(SKA, SVA, SKB, SVB, jnp.uint32(0), SV, ST, 1)
    radix_pass(SKB, SVB, SKA, SVA, jnp.uint32(8), SV, ST, 1)
    radix_pass(SKA, SVA, SKB, SVB, jnp.uint32(16), SV, ST, 1)
    radix_pass(SKB, SVB, SKA, SVA, jnp.uint32(24), SV, ST, 1)

    sbase = wid * ST
    pltpu.sync_copy(SKA.at[pl.ds(sbase, ST)], csk)
    pltpu.sync_copy(SVA.at[pl.ds(sbase, ST)], csv)
    g1 = pltpu.make_async_copy(X1.at[csv], px1g, sem)
    g2 = pltpu.make_async_copy(Y1.at[csv], py1g, sem)
    g3 = pltpu.make_async_copy(WW.at[csv], pwg, sem)
    g4 = pltpu.make_async_copy(HH.at[csv], phg, sem)
    g5 = pltpu.make_async_copy(X2.at[csv], px2g, sem)
    g6 = pltpu.make_async_copy(Y2.at[csv], py2g, sem)
    g7 = pltpu.make_async_copy(AR.at[csv], parg, sem)
    g8 = pltpu.make_async_copy(SS.at[csv], pscg, sem)
    for g in (g1, g2, g3, g4, g5, g6, g7, g8):
        g.start()
    for g in (g1, g2, g3, g4, g5, g6, g7, g8):
        g.wait()
    b2_outs = [
        pltpu.make_async_copy(px1g, PX1.at[pl.ds(sbase, ST)], sem),
        pltpu.make_async_copy(py1g, PY1.at[pl.ds(sbase, ST)], sem),
        pltpu.make_async_copy(pwg, PW.at[pl.ds(sbase, ST)], sem),
        pltpu.make_async_copy(phg, PH.at[pl.ds(sbase, ST)], sem),
        pltpu.make_async_copy(px2g, PX2.at[pl.ds(sbase, ST)], sem),
        pltpu.make_async_copy(py2g, PY2.at[pl.ds(sbase, ST)], sem),
        pltpu.make_async_copy(parg, PAR.at[pl.ds(sbase, ST)], sem),
        pltpu.make_async_copy(pscg, PS.at[pl.ds(sbase, ST)], sem),
    ]
    for g in b2_outs:
        g.start()
    for g in b2_outs:
        g.wait()

    def y_body(i, _):
        o = i * 16
        pos = sbase + o + _iota16()
        kk = plsc.bitcast(csk[pl.ds(o, 16)], jnp.uint32)
        valid = (kk != KEY_INVALID) & (pos < M)
        y2b = plsc.bitcast(py2g[pl.ds(o, 16)], jnp.int32)
        kv[pl.ds(o, 16)] = jnp.where(valid, y2b,
                                     jnp.full((16,), Y2_INVALID, jnp.int32))
        vv[pl.ds(o, 16)] = pos
        return _

    lax.fori_loop(0, SV, y_body, 0)
    pltpu.sync_copy(kv.at[pl.ds(0, ST)], CK.at[pl.ds(sbase, ST)])
    pltpu.sync_copy(vv.at[pl.ds(0, ST)], CV.at[pl.ds(sbase, ST)])
    plsc.subcore_barrier()

    radix_pass(CK, CV, NK, NV, jnp.uint32(0), SV, ST, 1)
    radix_pass(NK, NV, CK, CV, jnp.uint32(8), SV, ST, 1)
    radix_pass(CK, CV, NK, NV, jnp.uint32(16), SV, ST, 1)
    radix_pass(NK, NV, CK, CV, jnp.uint32(24), SV, ST, 1)

    pltpu.sync_copy(CK.at[pl.ds(sbase, ST)], csk)
    pltpu.sync_copy(CV.at[pl.ds(sbase, ST)], idxsv)
    pltpu.sync_copy(PX1, cpx1)
    pltpu.sync_copy(PY1, cpy1)
    pltpu.sync_copy(PX2, cpw)
    pltpu.sync_copy(PY2, cph)

    def c2_body(i, _):
        o = i * 16
        ix = idxsv[pl.ds(o, 16)]
        x1s[pl.ds(o, 16)] = plsc.load_gather(cpx1, [ix])
        y1s[pl.ds(o, 16)] = plsc.load_gather(cpy1, [ix])
        x2s[pl.ds(o, 16)] = plsc.load_gather(cpw, [ix])
        y2s[pl.ds(o, 16)] = plsc.load_gather(cph, [ix])
        alivev[pl.ds(o, 16)] = jnp.where(csk[pl.ds(o, 16)] != Y2_INVALID,
                                         jnp.full((16,), 1, jnp.int32),
                                         jnp.zeros((16,), jnp.int32))
        return _

    lax.fori_loop(0, SV, c2_body, 0)
    pltpu.sync_copy(x1s, X1S.at[pl.ds(sbase, ST)])
    pltpu.sync_copy(y1s, Y1S.at[pl.ds(sbase, ST)])
    pltpu.sync_copy(x2s, X2S.at[pl.ds(sbase, ST)])
    pltpu.sync_copy(y2s, Y2S.at[pl.ds(sbase, ST)])

    @pl.when(wid == 0)
    def _():
        for j in range(32):
            keepb[pl.ds(j * 16, 16)] = jnp.zeros((16,), jnp.int32)

    plsc.subcore_barrier()
    c2_dmas = [
        pltpu.make_async_copy(X1S, fx1, sem),
        pltpu.make_async_copy(Y1S, fy1, sem),
        pltpu.make_async_copy(X2S, fx2, sem),
        pltpu.make_async_copy(Y2S, fy2, sem),
        pltpu.make_async_copy(PAR, fden, sem),
        pltpu.make_async_copy(PAR.at[pl.ds(sbase, ST)], denv, sem),
    ]
    for g in c2_dmas:
        g.start()
    for g in c2_dmas:
        g.wait()

    @pl.when(wid == 0)
    def _():
        pltpu.sync_copy(CV.at[pl.ds(0, MP)], fidx)

    def nms_cond(st):
        c, cnt = st
        return (c >= 0) & (cnt < POST)

    def nms_body(st):
        c, cnt = st
        owner = c // 8
        lo = (c % 8) * 16
        co = c * 16

        @pl.when(wid == owner)
        def _():
            av = alivev[pl.ds(lo, 16)]
            x1c = x1s[pl.ds(lo, 16)]
            y1c = y1s[pl.ds(lo, 16)]
            x2c = x2s[pl.ds(lo, 16)]
            y2c = y2s[pl.ds(lo, 16)]
            dc = denv[pl.ds(lo, 16)]
            lanes = _iota16()

            def r_body(l_, carry):
                av_, km = carry
                l = 15 - l_
                al = _extract(av_, l) > 0
                bidx = jnp.zeros((16,), jnp.int32) + (co + l)
                bx1 = plsc.load_gather(fx1, [bidx])
                by1 = plsc.load_gather(fy1, [bidx])
                bx2 = plsc.load_gather(fx2, [bidx])
                xx1 = jnp.maximum(x1c, bx1)
                yy1 = jnp.maximum(y1c, by1)
                xx2 = jnp.minimum(x2c, bx2)
                wd = jnp.maximum(xx2 - xx1 + 1.0, 0.0)
                hd = jnp.maximum(y2c - yy1 + 1.0, 0.0)
                sup = (wd * hd) / dc >= THRESH
                is_l = lanes == l
                nav = jnp.where(sup | is_l, jnp.zeros((16,), jnp.int32), av_)
                av_n = jnp.where(al, nav, av_)
                km_n = jnp.where(al & is_l, jnp.full((16,), 1, jnp.int32), km)
                return av_n, km_n

            av_f, km_f = lax.fori_loop(0, 16, r_body,
                                       (av, jnp.zeros((16,), jnp.int32)))
            alivev[pl.ds(lo, 16)] = av_f
            tv16[...] = km_f
            pltpu.sync_copy(tv16, CHMA.at[pl.ds(co, 16)])

        plsc.subcore_barrier()
        pltpu.sync_copy(CHMA.at[pl.ds(co, 16)], tv16)
        km = tv16[...]
        kcnt = jnp.sum(km)

        @pl.when(kcnt > 0)
        def _():
            def ap_cond(mrem):
                return jnp.sum(mrem) > 0

            def ap_body(mrem):
                l0 = jnp.max(plsc.all_reduce_ffs(mrem != 0))
                bidx = jnp.zeros((16,), jnp.int32) + (co + l0)
                bx1 = plsc.load_gather(fx1, [bidx])
                by1 = plsc.load_gather(fy1, [bidx])
                bx2 = plsc.load_gather(fx2, [bidx])

                def s_body(v, _):
                    o = v * 16
                    xx1 = jnp.maximum(x1s[pl.ds(o, 16)], bx1)
                    yy1 = jnp.maximum(y1s[pl.ds(o, 16)], by1)
                    xx2 = jnp.minimum(x2s[pl.ds(o, 16)], bx2)
                    wd = jnp.maximum(xx2 - xx1 + 1.0, 0.0)
                    hd = jnp.maximum(y2s[pl.ds(o, 16)] - yy1 + 1.0, 0.0)
                    sup = (wd * hd) / denv[pl.ds(o, 16)] >= THRESH
                    alivev[pl.ds(o, 16)] = jnp.where(
                        sup, jnp.zeros((16,), jnp.int32), alivev[pl.ds(o, 16)])
                    return _

                lax.fori_loop(0, SV, s_body, 0)
                return jnp.where(_iota16() == l0, jnp.zeros((16,), jnp.int32),
                                 mrem)

            lax.while_loop(ap_cond, ap_body, km)

        @pl.when((wid == 0) & (kcnt > 0))
        def _():
            idxc = fidx[pl.ds(co, 16)]
            rincl = lax.rev(plsc.cumsum(lax.rev(km, (0,))), (0,))
            slot = cnt + rincl - 1
            plsc.store_scatter(keepb, [slot], idxc, mask=km != 0)

        return c - 1, cnt + kcnt

    _, cnt_f = lax.while_loop(nms_cond, nms_body,
                              (jnp.int32(MP // 16 - 1), jnp.int32(0)))

    @pl.when((wid == 0) & (cid == 0))
    def _():
        pltpu.sync_copy(PX1, cpx1)
        pltpu.sync_copy(PY1, cpy1)
        pltpu.sync_copy(PW, cpw)
        pltpu.sync_copy(PH, cph)
        pltpu.sync_copy(PS, cps)

        def e_body(j, _):
            o = j * 16
            jl = o + _iota16()
            kp = keepb[pl.ds(o, 16)]
            valid = (jl < cnt_f) & (jl < POST)
            zf = jnp.zeros((16,), jnp.float32)
            gx1 = jnp.where(valid, plsc.load_gather(cpx1, [kp]), zf)
            gy1 = jnp.where(valid, plsc.load_gather(cpy1, [kp]), zf)
            gw = jnp.where(valid, plsc.load_gather(cpw, [kp]), zf)
            gh = jnp.where(valid, plsc.load_gather(cph, [kp]), zf)
            gs = jnp.where(valid, plsc.load_gather(cps, [kp]), zf)
            plsc.store_scatter(pbuf, [jl * 4], gx1)
            plsc.store_scatter(pbuf, [jl * 4 + 1], gy1)
            plsc.store_scatter(pbuf, [jl * 4 + 2], gw)
            plsc.store_scatter(pbuf, [jl * 4 + 3], gh)
            sbuf[pl.ds(o, 16)] = gs
            return _

        lax.fori_loop(0, 19, e_body, 0)
        pltpu.sync_copy(pbuf.at[pl.ds(0, 1200)], op_hbm)
        pltpu.sync_copy(sbuf, os_hbm)


def _make_sc_call():
    mesh = plsc.VectorSubcoreMesh(core_axis_name="c", subcore_axis_name="s")
    f32 = jnp.float32
    i32 = jnp.int32
    vmem = [
        pltpu.VMEM((PT,), f32),
        pltpu.VMEM((PT * 4,), f32),
        pltpu.VMEM((PT * 4,), f32),
        pltpu.VMEM((PT,), f32),
        pltpu.VMEM((PT,), f32),
        pltpu.VMEM((PT,), f32),
        pltpu.VMEM((PT,), f32),
        pltpu.VMEM((PT,), f32),
        pltpu.VMEM((PT,), f32),
        pltpu.VMEM((PT,), f32),
        pltpu.VMEM((PT,), f32),
        pltpu.VMEM((1280,), i32),
        pltpu.VMEM((1280,), i32),
        pltpu.VMEM((1280,), i32),
        pltpu.VMEM((1280,), i32),
        pltpu.VMEM((PT,), i32),
        pltpu.VMEM((10, 128), i32),
        pltpu.VMEM((256,), i32),
        pltpu.VMEM((4096,), i32),
        pltpu.VMEM((256,), i32),
        pltpu.VMEM((256,), i32),
        pltpu.VMEM((ST,), i32),
        pltpu.VMEM((ST,), i32),
        pltpu.VMEM((ST,), f32),
        pltpu.VMEM((ST,), f32),
        pltpu.VMEM((ST,), f32),
        pltpu.VMEM((ST,), f32),
        pltpu.VMEM((ST,), f32),
        pltpu.VMEM((ST,), f32),
        pltpu.VMEM((ST,), f32),
        pltpu.VMEM((ST,), f32),
        pltpu.VMEM((ST,), f32),
        pltpu.VMEM((ST,), f32),
        pltpu.VMEM((ST,), f32),
        pltpu.VMEM((ST,), f32),
        pltpu.VMEM((ST,), f32),
        pltpu.VMEM((ST,), i32),
        pltpu.VMEM((ST,), i32),
        pltpu.VMEM((MP,), f32),
        pltpu.VMEM((MP,), f32),
        pltpu.VMEM((MP,), f32),
        pltpu.VMEM((MP,), f32),
        pltpu.VMEM((MP,), f32),
        pltpu.VMEM((MP,), f32),
        pltpu.VMEM((MP,), f32),
        pltpu.VMEM((MP,), f32),
        pltpu.VMEM((MP,), f32),
        pltpu.VMEM((MP,), f32),
        pltpu.VMEM((MP,), i32),
        pltpu.VMEM((512,), i32),
        pltpu.VMEM((16,), i32),
        pltpu.VMEM((1216,), f32),
        pltpu.VMEM((304,), f32),
    ]
    shared = [
        pltpu.VMEM_SHARED((MP + 128,), i32),
        pltpu.VMEM_SHARED((MP + 128,), i32),
        pltpu.VMEM_SHARED((MP + 128,), i32),
        pltpu.VMEM_SHARED((MP + 128,), i32),
        pltpu.VMEM_SHARED((NPAD,), f32),
        pltpu.VMEM_SHARED((NPAD,), f32),
        pltpu.VMEM_SHARED((NPAD,), f32),
        pltpu.VMEM_SHARED((NPAD,), f32),
        pltpu.VMEM_SHARED((NPAD,), f32),
        pltpu.VMEM_SHARED((NPAD,), f32),
        pltpu.VMEM_SHARED((NPAD,), f32),
        pltpu.VMEM_SHARED((NPAD,), f32),
        pltpu.VMEM_SHARED((4096,), i32),
        pltpu.VMEM_SHARED((16384,), i32),
        pltpu.VMEM_SHARED((256,), i32),
        pltpu.VMEM_SHARED((MP,), f32),
        pltpu.VMEM_SHARED((MP,), f32),
        pltpu.VMEM_SHARED((MP,), f32),
        pltpu.VMEM_SHARED((MP,), f32),
        pltpu.VMEM_SHARED((MP,), f32),
        pltpu.VMEM_SHARED((MP,), f32),
        pltpu.VMEM_SHARED((MP,), f32),
        pltpu.VMEM_SHARED((MP,), f32),
        pltpu.VMEM_SHARED((MP + 128,), i32),
        pltpu.VMEM_SHARED((MP + 128,), i32),
        pltpu.VMEM_SHARED((MP + 128,), i32),
        pltpu.VMEM_SHARED((MP + 128,), i32),
        pltpu.VMEM_SHARED((MP,), f32),
        pltpu.VMEM_SHARED((MP,), f32),
        pltpu.VMEM_SHARED((MP,), f32),
        pltpu.VMEM_SHARED((MP,), f32),
        pltpu.VMEM_SHARED((MP,), i32),
    ]
    return pl.kernel(
        _sc_body,
        out_type=(jax.ShapeDtypeStruct((1200,), jnp.float32),
                  jax.ShapeDtypeStruct((304,), jnp.float32)),
        mesh=mesh,
        compiler_params=pltpu.CompilerParams(needs_layout_passes=False),
        scratch_types=vmem + shared + [pltpu.SemaphoreType.DMA],
    )


_sc_call = _make_sc_call()


@jax.jit
def kernel(scores, bbox_deltas, image_metadata, boxes):
    del image_metadata
    sc = jnp.reshape(scores, (-1,))
    dl = jnp.reshape(bbox_deltas, (-1,))
    bx = jnp.reshape(boxes, (-1,))
    props_flat, scs_flat = _sc_call(sc, dl, bx)
    proposal_outputs = jnp.reshape(props_flat, (1, POST, 4))
    score_outputs = jnp.reshape(scs_flat[:POST], (1, POST, 1))
    return proposal_outputs, score_outputs


# speedup vs baseline: 1.0730x; 1.0017x over previous
import functools
import jax
import jax.numpy as jnp
import numpy as np
from jax import lax
from jax.experimental import pallas as pl
from jax.experimental.pallas import tpu as pltpu
from jax.experimental.pallas import tpu_sc as plsc

N = 19881
NT = 16
NPAD = 19968
PT = NPAD // NT
PV = PT // 16
M = 2000
MP = 2048
ST = MP // NT
SV = ST // 16
POST = 300
THRESH = 0.7
CLIP = 46.0
KEY_INVALID = np.uint32(0xFFFFFFFF)
Y2_INVALID = np.int32(0x7F800000)
DUMP = NPAD


def _iota16():
    return lax.iota(jnp.int32, 16)


def _extract(vec, lane):
    return jnp.sum(jnp.where(_iota16() == lane, vec, jnp.zeros((16,), vec.dtype)))


def _sc_body(sc_hbm, dl_hbm, bx_hbm, op_hbm, os_hbm,
             in_s, in_d, in_b,
             lx1, ly1, lw, lh, lx2, ly2, lar, lsc, lkey, lval,
             kv, vv, rankv, destv, hist, ha, gbase, tbase,
             csk, csv, px1g, py1g, pwg, phg, px2g, py2g, parg, pscg,
             x1s, y1s, x2s, y2s, denv, alivev, idxsv,
             cpx1, cpy1, cpw, cph, cps,
             fx1, fy1, fx2, fy2, fden, fidx,
             keepb, tv16, pbuf, sbuf,
             CK, CV, NK, NV, X1, Y1, WW, HH, X2, Y2, AR, SS,
             HIST, HISTS, CNTS, PX1, PY1, PW, PH, PX2, PY2, PAR, PS,
             SKA, SVA, SKB, SVB,
             X1S, Y1S, X2S, Y2S, CHMA,
             sem):
    cid = lax.axis_index("c")
    wid = lax.axis_index("s")
    base = wid * PT

    TAIL = N - 15 * PT

    @pl.when(wid < 15)
    def _():
        ins = [
            pltpu.make_async_copy(sc_hbm.at[pl.ds(base, PT)], in_s, sem),
            pltpu.make_async_copy(dl_hbm.at[pl.ds(base * 4, PT * 4)], in_d, sem),
            pltpu.make_async_copy(bx_hbm.at[pl.ds(base * 4, PT * 4)], in_b, sem),
        ]
        for g in ins:
            g.start()
        for g in ins:
            g.wait()

    @pl.when(wid == 15)
    def _():
        b15 = 15 * PT
        ins = [
            pltpu.make_async_copy(sc_hbm.at[pl.ds(b15, TAIL)],
                                  in_s.at[pl.ds(0, TAIL)], sem),
            pltpu.make_async_copy(dl_hbm.at[pl.ds(b15 * 4, TAIL * 4)],
                                  in_d.at[pl.ds(0, TAIL * 4)], sem),
            pltpu.make_async_copy(bx_hbm.at[pl.ds(b15 * 4, TAIL * 4)],
                                  in_b.at[pl.ds(0, TAIL * 4)], sem),
        ]
        for g in ins:
            g.start()
        for g in ins:
            g.wait()

    def a_body(i, _):
        o = i * 16
        lanes = _iota16()
        q = (o + lanes) * 4
        d0 = plsc.load_gather(in_d, [q])
        d1 = plsc.load_gather(in_d, [q + 1])
        d2 = plsc.load_gather(in_d, [q + 2])
        d3 = plsc.load_gather(in_d, [q + 3])
        b0 = plsc.load_gather(in_b, [q])
        b1 = plsc.load_gather(in_b, [q + 1])
        b2 = plsc.load_gather(in_b, [q + 2])
        b3 = plsc.load_gather(in_b, [q + 3])
        s = in_s[pl.ds(o, 16)]
        ax = d0 * b0 + b0
        ay = d1 * b1 + b1
        aw = jnp.exp(d2) * b2
        ah = jnp.exp(d3) * b3
        zero = jnp.float32(0.0)
        ax = jnp.maximum(ax, zero)
        ay = jnp.maximum(ay, zero)
        aw = jnp.maximum(aw, zero)
        ah = jnp.maximum(ah, zero)
        x1 = jnp.minimum(ax, CLIP)
        y1 = jnp.minimum(ay, CLIP)
        x2 = jnp.minimum(ax + aw - 1.0, CLIP)
        y2 = jnp.minimum(ay + ah - 1.0, CLIP)
        w_ = x2 - x1 + 1.0
        h_ = y2 - y1 + 1.0
        e = base + o + lanes
        keep = (w_ >= 2.0) & (h_ >= 2.0) & (e < N)
        u = plsc.bitcast(s + zero, jnp.uint32)
        asc = jnp.where((u >> 31) == jnp.uint32(1), ~u, u | jnp.uint32(0x80000000))
        keyd = jnp.where(keep, ~asc, KEY_INVALID)
        lx1[pl.ds(o, 16)] = x1
        ly1[pl.ds(o, 16)] = y1
        lw[pl.ds(o, 16)] = w_
        lh[pl.ds(o, 16)] = h_
        lx2[pl.ds(o, 16)] = x2
        ly2[pl.ds(o, 16)] = y2
        lar[pl.ds(o, 16)] = w_ * h_
        lsc[pl.ds(o, 16)] = s
        lkey[pl.ds(o, 16)] = plsc.bitcast(keyd, jnp.int32)
        lval[pl.ds(o, 16)] = e
        return _

    lax.fori_loop(0, PV, a_body, 0)
    a_outs = [
        pltpu.make_async_copy(lx1, X1.at[pl.ds(base, PT)], sem),
        pltpu.make_async_copy(ly1, Y1.at[pl.ds(base, PT)], sem),
        pltpu.make_async_copy(lw, WW.at[pl.ds(base, PT)], sem),
        pltpu.make_async_copy(lh, HH.at[pl.ds(base, PT)], sem),
        pltpu.make_async_copy(lx2, X2.at[pl.ds(base, PT)], sem),
        pltpu.make_async_copy(ly2, Y2.at[pl.ds(base, PT)], sem),
        pltpu.make_async_copy(lar, AR.at[pl.ds(base, PT)], sem),
        pltpu.make_async_copy(lsc, SS.at[pl.ds(base, PT)], sem),
    ]
    for g in a_outs:
        g.start()
    for g in a_outs:
        g.wait()

    P = jnp.uint32(0)
    R = jnp.int32(M)
    for rnd, shift in enumerate((24, 16, 8, 0)):
        dmask = jnp.uint32((0xFFFFFFFF00000000 >> (8 * rnd)) & 0xFFFFFFFF)
        hb = rnd * 4096

        for j in range(16):
            hist[pl.ds(j * 16, 16)] = jnp.zeros((16,), jnp.int32)

        def s_hist(i, _, shift=shift, dmask=dmask, P=P):
            o = i * 16
            ku = plsc.bitcast(lkey[pl.ds(o, 16)], jnp.uint32)
            m = (ku & dmask) == (P & dmask)
            d = ((ku >> jnp.uint32(shift)) & jnp.uint32(255)).astype(jnp.int32)
            cnt, last = plsc.scan_count(d, mask=m)
            plsc.addupdate_scatter(hist, [d], cnt, mask=last)
            return _

        lax.fori_loop(0, PV, s_hist, 0)
        pltpu.sync_copy(hist, HISTS.at[pl.ds(hb + wid * 256, 256)])
        plsc.subcore_barrier()
        pltpu.sync_copy(HISTS.at[pl.ds(hb, 4096)], ha)

        def s_find(j, carry):
            found, bsel, rminus, cumbefore = carry
            o = j * 16
            tot = jnp.zeros((16,), jnp.int32)
            for t in range(16):
                tot = tot + ha[pl.ds(t * 256 + o, 16)]
            incl = plsc.cumsum(tot)
            cum = cumbefore + incl
            l = jnp.max(plsc.all_reduce_ffs(cum >= R))
            this = (l < 16) & jnp.logical_not(found)
            bsel = jnp.where(this, o + l, bsel)
            rminus = jnp.where(this,
                               cumbefore + _extract(incl, l) - _extract(tot, l),
                               rminus)
            return (found | (l < 16), bsel, rminus,
                    cumbefore + jnp.sum(tot))

        _, bsel, rminus, _ = lax.fori_loop(
            0, 16, s_find,
            (jnp.bool_(False), jnp.int32(0), jnp.int32(0), jnp.int32(0)))
        P = P | (bsel.astype(jnp.uint32) << jnp.uint32(shift))
        R = R - rminus
    T = P

    def cnt_body(i, carry):
        nlt, nt = carry
        o = i * 16
        ku = plsc.bitcast(lkey[pl.ds(o, 16)], jnp.uint32)
        one = jnp.full((16,), 1, jnp.int32)
        z = jnp.zeros((16,), jnp.int32)
        return (nlt + jnp.sum(jnp.where(ku < T, one, z)),
                nt + jnp.sum(jnp.where(ku == T, one, z)))

    nlt, ntie = lax.fori_loop(0, PV, cnt_body, (jnp.int32(0), jnp.int32(0)))
    lanes0 = _iota16()
    tv16[...] = (jnp.where(lanes0 == 0, nlt, jnp.zeros((16,), jnp.int32))
                 + jnp.where(lanes0 == 1, ntie, jnp.zeros((16,), jnp.int32)))
    pltpu.sync_copy(tv16, CNTS.at[pl.ds(wid * 16, 16)])
    plsc.subcore_barrier()
    pltpu.sync_copy(CNTS, ha.at[pl.ds(0, 256)])

    def base_body(t, carry):
        blt, btie, c1 = carry
        row = ha[pl.ds(t * 16, 16)]
        nlt_t = _extract(row, 0)
        nt_t = _extract(row, 1)
        zero = jnp.int32(0)
        return (blt + jnp.where(t < wid, nlt_t, zero),
                btie + jnp.where(t < wid, nt_t, zero),
                c1 + nlt_t)

    blt, btie, c1 = lax.fori_loop(0, 16, base_body,
                                  (jnp.int32(0), jnp.int32(0), jnp.int32(0)))

    def dest_rows():
        def d2_body(i, carry):
            lt_run, tie_run = carry
            o = i * 16
            ku = plsc.bitcast(lkey[pl.ds(o, 16)], jnp.uint32)
            mlt = ku < T
            meq = ku == T
            one = jnp.full((16,), 1, jnp.int32)
            z = jnp.zeros((16,), jnp.int32)
            ilt = jnp.where(mlt, one, z)
            ieq = jnp.where(meq, one, z)
            ex_lt = plsc.cumsum(ilt) - ilt
            ex_tie = plsc.cumsum(ieq) - ieq
            d_lt = blt + lt_run + ex_lt
            g = btie + tie_run + ex_tie
            d_tie = c1 + g
            dump = MP + ((o + _iota16()) % 128)
            dest = jnp.where(mlt, d_lt,
                             jnp.where(meq & (g < R), d_tie, dump))
            r_ = i // 8
            destv[r_, pl.ds((i % 8) * 16, 16)] = dest
            return lt_run + jnp.sum(ilt), tie_run + jnp.sum(ieq)

        lax.fori_loop(0, PV, d2_body, (jnp.int32(0), jnp.int32(0)))
        for i in range(PV, 80):
            destv[i // 8, pl.ds((i % 8) * 16, 16)] = MP + (i % 8) * 16 + _iota16()

    dest_rows()
    s2_dmas = []
    for r in range(10):
        s2_dmas.append(pltpu.make_async_copy(lkey.at[pl.ds(r * 128, 128)],
                                             SKA.at[destv.at[r]], sem))
        s2_dmas.append(pltpu.make_async_copy(lval.at[pl.ds(r * 128, 128)],
                                             SVA.at[destv.at[r]], sem))
    for g in s2_dmas:
        g.start()
    for g in s2_dmas:
        g.wait()

    @pl.when(wid == 0)
    def _():
        for j in range(3):
            tv16[...] = jnp.full((16,), -1, jnp.int32)
            pltpu.sync_copy(tv16, SKA.at[pl.ds(M + j * 16, 16)])
            tv16[...] = jnp.zeros((16,), jnp.int32)
            pltpu.sync_copy(tv16, SVA.at[pl.ds(M + j * 16, 16)])

    plsc.subcore_barrier()

    def radix_pass(src_k, src_v, dst_k, dst_v, shift, nv, per_tile, nrows):
        b0 = wid * per_tile
        rp_ins = [
            pltpu.make_async_copy(src_k.at[pl.ds(b0, per_tile)],
                                  kv.at[pl.ds(0, per_tile)], sem),
            pltpu.make_async_copy(src_v.at[pl.ds(b0, per_tile)],
                                  vv.at[pl.ds(0, per_tile)], sem),
        ]
        for g in rp_ins:
            g.start()
        for g in rp_ins:
            g.wait()
        for j in range(16):
            hist[pl.ds(j * 16, 16)] = jnp.zeros((16,), jnp.int32)

        def h_body(i, _):
            o = i * 16
            ku = plsc.bitcast(kv[pl.ds(o, 16)], jnp.uint32)
            d = ((ku >> shift) & jnp.uint32(255)).astype(jnp.int32)
            pre = plsc.load_gather(hist, [d])
            cnt, last = plsc.scan_count(d)
            rankv[pl.ds(o, 16)] = pre + cnt - 1
            plsc.addupdate_scatter(hist, [d], cnt, mask=last)
            return _

        lax.fori_loop(0, nv, h_body, 0)
        pltpu.sync_copy(hist, HIST.at[pl.ds(wid * 256, 256)])
        plsc.subcore_barrier()
        pltpu.sync_copy(HIST, ha)

        def g_body(j, carry):
            o = j * 16
            tot = jnp.zeros((16,), jnp.int32)
            tb = jnp.zeros((16,), jnp.int32)
            for t in range(16):
                row = ha[pl.ds(t * 256 + o, 16)]
                tot = tot + row
                tb = tb + jnp.where(jnp.int32(t) < wid, row,
                                    jnp.zeros((16,), jnp.int32))
            incl = plsc.cumsum(tot)
            gbase[pl.ds(o, 16)] = incl - tot + carry
            tbase[pl.ds(o, 16)] = tb
            return carry + jnp.sum(tot)

        lax.fori_loop(0, 16, g_body, jnp.int32(0))

        for r in range(nrows):
            vlo = r * 8
            vhi = min(r * 8 + 8, nv)

            def d_body(i, _, vlo=vlo, r=r):
                o = i * 16
                ku = plsc.bitcast(kv[pl.ds(vlo * 16 + o, 16)], jnp.uint32)
                d = ((ku >> shift) & jnp.uint32(255)).astype(jnp.int32)
                gb = plsc.load_gather(gbase, [d])
                tb = plsc.load_gather(tbase, [d])
                dst = gb + tb + rankv[pl.ds(vlo * 16 + o, 16)]
                destv[r, pl.ds(o, 16)] = dst
                return _

            lax.fori_loop(0, vhi - vlo, d_body, 0)
            for i in range(vhi - vlo, 8):
                destv[r, pl.ds(i * 16, 16)] = DUMP + i * 16 + _iota16()
        rp_outs = []
        for r in range(nrows):
            rp_outs.append(pltpu.make_async_copy(kv.at[pl.ds(r * 128, 128)],
                                                 dst_k.at[destv.at[r]], sem))
            rp_outs.append(pltpu.make_async_copy(vv.at[pl.ds(r * 128, 128)],
                                                 dst_v.at[destv.at[r]], sem))
        for g in rp_outs:
            g.start()
        for g in rp_outs:
            g.wait()
        plsc.subcore_barrier()

    radix_pass(SKA, SVA, SKB, SVB, jnp.uint32(0), SV, ST, 1)
    radix_pass(SKB, SVB, SKA, SVA, jnp.uint32(8), SV, ST, 1)
    radix_pass(SKA, SVA, SKB, SVB, jnp.uint32(16), SV, ST, 1)
    radix_pass(SKB, SVB, SKA, SVA, jnp.uint32(24), SV, ST, 1)

    sbase = wid * ST
    pltpu.sync_copy(SKA.at[pl.ds(sbase, ST)], csk)
    pltpu.sync_copy(SVA.at[pl.ds(sbase, ST)], csv)
    g1 = pltpu.make_async_copy(X1.at[csv], px1g, sem)
    g2 = pltpu.make_async_copy(Y1.at[csv], py1g, sem)
    g3 = pltpu.make_async_copy(WW.at[csv], pwg, sem)
    g4 = pltpu.make_async_copy(HH.at[csv], phg, sem)
    g5 = pltpu.make_async_copy(X2.at[csv], px2g, sem)
    g6 = pltpu.make_async_copy(Y2.at[csv], py2g, sem)
    g7 = pltpu.make_async_copy(AR.at[csv], parg, sem)
    g8 = pltpu.make_async_copy(SS.at[csv], pscg, sem)
    for g in (g1, g2, g3, g4, g5, g6, g7, g8):
        g.start()
    for g in (g1, g2, g3, g4, g5, g6, g7, g8):
        g.wait()
    b2_outs = [
        pltpu.make_async_copy(px1g, PX1.at[pl.ds(sbase, ST)], sem),
        pltpu.make_async_copy(py1g, PY1.at[pl.ds(sbase, ST)], sem),
        pltpu.make_async_copy(pwg, PW.at[pl.ds(sbase, ST)], sem),
        pltpu.make_async_copy(phg, PH.at[pl.ds(sbase, ST)], sem),
        pltpu.make_async_copy(px2g, PX2.at[pl.ds(sbase, ST)], sem),
        pltpu.make_async_copy(py2g, PY2.at[pl.ds(sbase, ST)], sem),
        pltpu.make_async_copy(parg, PAR.at[pl.ds(sbase, ST)], sem),
        pltpu.make_async_copy(pscg, PS.at[pl.ds(sbase, ST)], sem),
    ]
    for g in b2_outs:
        g.start()
    for g in b2_outs:
        g.wait()

    def y_body(i, _):
        o = i * 16
        pos = sbase + o + _iota16()
        kk = plsc.bitcast(csk[pl.ds(o, 16)], jnp.uint32)
        valid = (kk != KEY_INVALID) & (pos < M)
        y2b = plsc.bitcast(py2g[pl.ds(o, 16)], jnp.int32)
        kv[pl.ds(o, 16)] = jnp.where(valid, y2b,
                                     jnp.full((16,), Y2_INVALID, jnp.int32))
        vv[pl.ds(o, 16)] = pos
        return _

    lax.fori_loop(0, SV, y_body, 0)
    pltpu.sync_copy(kv.at[pl.ds(0, ST)], CK.at[pl.ds(sbase, ST)])
    pltpu.sync_copy(vv.at[pl.ds(0, ST)], CV.at[pl.ds(sbase, ST)])
    plsc.subcore_barrier()

    radix_pass(CK, CV, NK, NV, jnp.uint32(0), SV, ST, 1)
    radix_pass(NK, NV, CK, CV, jnp.uint32(8), SV, ST, 1)
    radix_pass(CK, CV, NK, NV, jnp.uint32(16), SV, ST, 1)
    radix_pass(NK, NV, CK, CV, jnp.uint32(24), SV, ST, 1)

    pltpu.sync_copy(CK.at[pl.ds(sbase, ST)], csk)
    pltpu.sync_copy(CV.at[pl.ds(sbase, ST)], idxsv)
    pltpu.sync_copy(PX1, cpx1)
    pltpu.sync_copy(PY1, cpy1)
    pltpu.sync_copy(PX2, cpw)
    pltpu.sync_copy(PY2, cph)

    def c2_body(i, _):
        o = i * 16
        ix = idxsv[pl.ds(o, 16)]
        x1s[pl.ds(o, 16)] = plsc.load_gather(cpx1, [ix])
        y1s[pl.ds(o, 16)] = plsc.load_gather(cpy1, [ix])
        x2s[pl.ds(o, 16)] = plsc.load_gather(cpw, [ix])
        y2s[pl.ds(o, 16)] = plsc.load_gather(cph, [ix])
        alivev[pl.ds(o, 16)] = jnp.where(csk[pl.ds(o, 16)] != Y2_INVALID,
                                         jnp.full((16,), 1, jnp.int32),
                                         jnp.zeros((16,), jnp.int32))
        return _

    lax.fori_loop(0, SV, c2_body, 0)
    pltpu.sync_copy(x1s, X1S.at[pl.ds(sbase, ST)])
    pltpu.sync_copy(y1s, Y1S.at[pl.ds(sbase, ST)])
    pltpu.sync_copy(x2s, X2S.at[pl.ds(sbase, ST)])
    pltpu.sync_copy(y2s, Y2S.at[pl.ds(sbase, ST)])

    @pl.when(wid == 0)
    def _():
        for j in range(32):
            keepb[pl.ds(j * 16, 16)] = jnp.zeros((16,), jnp.int32)

    plsc.subcore_barrier()
    c2_dmas = [
        pltpu.make_async_copy(X1S, fx1, sem),
        pltpu.make_async_copy(Y1S, fy1, sem),
        pltpu.make_async_copy(X2S, fx2, sem),
        pltpu.make_async_copy(Y2S, fy2, sem),
        pltpu.make_async_copy(PAR, fden, sem),
        pltpu.make_async_copy(PAR.at[pl.ds(sbase, ST)], denv, sem),
    ]
    for g in c2_dmas:
        g.start()
    for g in c2_dmas:
        g.wait()

    @pl.when(wid == 0)
    def _():
        pltpu.sync_copy(CV.at[pl.ds(0, MP)], fidx)

    def nms_cond(st):
        c, cnt = st
        return (c >= 0) & (cnt < POST)

    def nms_body(st):
        c, cnt = st
        owner = c // 8
        lo = (c % 8) * 16
        co = c * 16

        @pl.when(wid == owner)
        def _():
            av = alivev[pl.ds(lo, 16)]
            x1c = x1s[pl.ds(lo, 16)]
            y1c = y1s[pl.ds(lo, 16)]
            x2c = x2s[pl.ds(lo, 16)]
            y2c = y2s[pl.ds(lo, 16)]
            dc = denv[pl.ds(lo, 16)]
            lanes = _iota16()

            def r_body(l_, carry):
                av_, km = carry
                l = 15 - l_
                al = _extract(av_, l) > 0
                bidx = jnp.zeros((16,), jnp.int32) + (co + l)
                bx1 = plsc.load_gather(fx1, [bidx])
                by1 = plsc.load_gather(fy1, [bidx])
                bx2 = plsc.load_gather(fx2, [bidx])
                xx1 = jnp.maximum(x1c, bx1)
                yy1 = jnp.maximum(y1c, by1)
                xx2 = jnp.minimum(x2c, bx2)
                wd = jnp.maximum(xx2 - xx1 + 1.0, 0.0)
                hd = jnp.maximum(y2c - yy1 + 1.0, 0.0)
                sup = (wd * hd) / dc >= THRESH
                is_l = lanes == l
                nav = jnp.where(sup | is_l, jnp.zeros((16,), jnp.int32), av_)
                av_n = jnp.where(al, nav, av_)
                km_n = jnp.where(al & is_l, jnp.full((16,), 1, jnp.int32), km)
                return av_n, km_n

            av_f, km_f = lax.fori_loop(0, 16, r_body,
                                       (av, jnp.zeros((16,), jnp.int32)))
            alivev[pl.ds(lo, 16)] = av_f
            tv16[...] = km_f
            pltpu.sync_copy(tv16, CHMA.at[pl.ds(co, 16)])

        plsc.subcore_barrier()
        pltpu.sync_copy(CHMA.at[pl.ds(co, 16)], tv16)
        km = tv16[...]
        kcnt = jnp.sum(km)

        @pl.when(kcnt > 0)
        def _():
            def ap_cond(mrem):
                return jnp.sum(mrem) > 0

            def ap_body(mrem):
                l0 = jnp.max(plsc.all_reduce_ffs(mrem != 0))
                bidx = jnp.zeros((16,), jnp.int32) + (co + l0)
                bx1 = plsc.load_gather(fx1, [bidx])
                by1 = plsc.load_gather(fy1, [bidx])
                bx2 = plsc.load_gather(fx2, [bidx])

                def s_body(v, _):
                    o = v * 16
                    xx1 = jnp.maximum(x1s[pl.ds(o, 16)], bx1)
                    yy1 = jnp.maximum(y1s[pl.ds(o, 16)], by1)
                    xx2 = jnp.minimum(x2s[pl.ds(o, 16)], bx2)
                    wd = jnp.maximum(xx2 - xx1 + 1.0, 0.0)
                    hd = jnp.maximum(y2s[pl.ds(o, 16)] - yy1 + 1.0, 0.0)
                    sup = (wd * hd) / denv[pl.ds(o, 16)] >= THRESH
                    alivev[pl.ds(o, 16)] = jnp.where(
                        sup, jnp.zeros((16,), jnp.int32), alivev[pl.ds(o, 16)])
                    return _

                lax.fori_loop(0, SV, s_body, 0)
                return jnp.where(_iota16() == l0, jnp.zeros((16,), jnp.int32),
                                 mrem)

            lax.while_loop(ap_cond, ap_body, km)

        @pl.when((wid == 0) & (kcnt > 0))
        def _():
            idxc = fidx[pl.ds(co, 16)]
            rincl = lax.rev(plsc.cumsum(lax.rev(km, (0,))), (0,))
            slot = cnt + rincl - 1
            plsc.store_scatter(keepb, [slot], idxc, mask=km != 0)

        return c - 1, cnt + kcnt

    _, cnt_f = lax.while_loop(nms_cond, nms_body,
                              (jnp.int32(MP // 16 - 1), jnp.int32(0)))

    @pl.when((wid == 0) & (cid == 0))
    def _():
        pltpu.sync_copy(PX1, cpx1)
        pltpu.sync_copy(PY1, cpy1)
        pltpu.sync_copy(PW, cpw)
        pltpu.sync_copy(PH, cph)
        pltpu.sync_copy(PS, cps)

        def e_body(j, _):
            o = j * 16
            jl = o + _iota16()
            kp = keepb[pl.ds(o, 16)]
            valid = (jl < cnt_f) & (jl < POST)
            zf = jnp.zeros((16,), jnp.float32)
            gx1 = jnp.where(valid, plsc.load_gather(cpx1, [kp]), zf)
            gy1 = jnp.where(valid, plsc.load_gather(cpy1, [kp]), zf)
            gw = jnp.where(valid, plsc.load_gather(cpw, [kp]), zf)
            gh = jnp.where(valid, plsc.load_gather(cph, [kp]), zf)
            gs = jnp.where(valid, plsc.load_gather(cps, [kp]), zf)
            plsc.store_scatter(pbuf, [jl * 4], gx1)
            plsc.store_scatter(pbuf, [jl * 4 + 1], gy1)
            plsc.store_scatter(pbuf, [jl * 4 + 2], gw)
            plsc.store_scatter(pbuf, [jl * 4 + 3], gh)
            sbuf[pl.ds(o, 16)] = gs
            return _

        lax.fori_loop(0, 19, e_body, 0)
        pltpu.sync_copy(pbuf.at[pl.ds(0, 1200)], op_hbm)
        pltpu.sync_copy(sbuf, os_hbm)


def _make_sc_call():
    mesh = plsc.VectorSubcoreMesh(core_axis_name="c", subcore_axis_name="s")
    f32 = jnp.float32
    i32 = jnp.int32
    vmem = [
        pltpu.VMEM((PT,), f32),
        pltpu.VMEM((PT * 4,), f32),
        pltpu.VMEM((PT * 4,), f32),
        pltpu.VMEM((PT,), f32),
        pltpu.VMEM((PT,), f32),
        pltpu.VMEM((PT,), f32),
        pltpu.VMEM((PT,), f32),
        pltpu.VMEM((PT,), f32),
        pltpu.VMEM((PT,), f32),
        pltpu.VMEM((PT,), f32),
        pltpu.VMEM((PT,), f32),
        pltpu.VMEM((1280,), i32),
        pltpu.VMEM((1280,), i32),
        pltpu.VMEM((1280,), i32),
        pltpu.VMEM((1280,), i32),
        pltpu.VMEM((PT,), i32),
        pltpu.VMEM((10, 128), i32),
        pltpu.VMEM((256,), i32),
        pltpu.VMEM((4096,), i32),
        pltpu.VMEM((256,), i32),
        pltpu.VMEM((256,), i32),
        pltpu.VMEM((ST,), i32),
        pltpu.VMEM((ST,), i32),
        pltpu.VMEM((ST,), f32),
        pltpu.VMEM((ST,), f32),
        pltpu.VMEM((ST,), f32),
        pltpu.VMEM((ST,), f32),
        pltpu.VMEM((ST,), f32),
        pltpu.VMEM((ST,), f32),
        pltpu.VMEM((ST,), f32),
        pltpu.VMEM((ST,), f32),
        pltpu.VMEM((ST,), f32),
        pltpu.VMEM((ST,), f32),
        pltpu.VMEM((ST,), f32),
        pltpu.VMEM((ST,), f32),
        pltpu.VMEM((ST,), f32),
        pltpu.VMEM((ST,), i32),
        pltpu.VMEM((ST,), i32),
        pltpu.VMEM((MP,), f32),
        pltpu.VMEM((MP,), f32),
        pltpu.VMEM((MP,), f32),
        pltpu.VMEM((MP,), f32),
        pltpu.VMEM((MP,), f32),
        pltpu.VMEM((MP,), f32),
        pltpu.VMEM((MP,), f32),
        pltpu.VMEM((MP,), f32),
        pltpu.VMEM((MP,), f32),
        pltpu.VMEM((MP,), f32),
        pltpu.VMEM((MP,), i32),
        pltpu.VMEM((512,), i32),
        pltpu.VMEM((16,), i32),
        pltpu.VMEM((1216,), f32),
        pltpu.VMEM((304,), f32),
    ]
    shared = [
        pltpu.VMEM_SHARED((MP + 128,), i32),
        pltpu.VMEM_SHARED((MP + 128,), i32),
        pltpu.VMEM_SHARED((MP + 128,), i32),
        pltpu.VMEM_SHARED((MP + 128,), i32),
        pltpu.VMEM_SHARED((NPAD,), f32),
        pltpu.VMEM_SHARED((NPAD,), f32),
        pltpu.VMEM_SHARED((NPAD,), f32),
        pltpu.VMEM_SHARED((NPAD,), f32),
        pltpu.VMEM_SHARED((NPAD,), f32),
        pltpu.VMEM_SHARED((NPAD,), f32),
        pltpu.VMEM_SHARED((NPAD,), f32),
        pltpu.VMEM_SHARED((NPAD,), f32),
        pltpu.VMEM_SHARED((4096,), i32),
        pltpu.VMEM_SHARED((16384,), i32),
        pltpu.VMEM_SHARED((256,), i32),
        pltpu.VMEM_SHARED((MP,), f32),
        pltpu.VMEM_SHARED((MP,), f32),
        pltpu.VMEM_SHARED((MP,), f32),
        pltpu.VMEM_SHARED((MP,), f32),
        pltpu.VMEM_SHARED((MP,), f32),
        pltpu.VMEM_SHARED((MP,), f32),
        pltpu.VMEM_SHARED((MP,), f32),
        pltpu.VMEM_SHARED((MP,), f32),
        pltpu.VMEM_SHARED((MP + 128,), i32),
        pltpu.VMEM_SHARED((MP + 128,), i32),
        pltpu.VMEM_SHARED((MP + 128,), i32),
        pltpu.VMEM_SHARED((MP + 128,), i32),
        pltpu.VMEM_SHARED((MP,), f32),
        pltpu.VMEM_SHARED((MP,), f32),
        pltpu.VMEM_SHARED((MP,), f32),
        pltpu.VMEM_SHARED((MP,), f32),
        pltpu.VMEM_SHARED((MP,), i32),
    ]
    return pl.kernel(
        _sc_body,
        out_type=(jax.ShapeDtypeStruct((1200,), jnp.float32),
                  jax.ShapeDtypeStruct((304,), jnp.float32)),
        mesh=mesh,
        compiler_params=pltpu.CompilerParams(needs_layout_passes=False),
        scratch_types=vmem + shared + [pltpu.SemaphoreType.DMA],
    )


_SC_CALL_CACHE = []


def kernel(scores, bbox_deltas, image_metadata, boxes):
    del image_metadata
    if not _SC_CALL_CACHE:
        _SC_CALL_CACHE.append(jax.jit(_make_sc_call()))
    sc_call = _SC_CALL_CACHE[0]
    sc = jnp.reshape(scores, (-1,))
    dl = jnp.reshape(bbox_deltas, (-1,))
    bx = jnp.reshape(boxes, (-1,))
    props_flat, scs_flat = sc_call(sc, dl, bx)
    proposal_outputs = jnp.reshape(props_flat, (1, POST, 4))
    score_outputs = jnp.reshape(scs_flat[:POST], (1, POST, 1))
    return proposal_outputs, score_outputs
